# Initial kernel scaffold; baseline (speedup 1.0000x reference)
import jax
import jax.numpy as jnp
from jax.experimental import pallas as pl


def kernel(X, v_hier, e_hier, v_cooc, e_cooc, v_cite, e_cite, W, b, gamma, beta, w_e, a1, a2, a3):
    raise NotImplementedError("write your pallas kernel here")


# speedup vs baseline: 4.5789x; 4.5789x over previous
import jax
import jax.numpy as jnp
from jax import lax
from jax.experimental import pallas as pl
from jax.experimental.pallas import tpu as pltpu
from jax.experimental.pallas import tpu_sc as plsc

N = 10000
D = 256
M = 5000
E = 160000

NTILES = 16
E_PAD = 163840
CHUNKS = E_PAD // NTILES // 128
CCHUNKS = CHUNKS // 2
M_PAD = 5120
N_PAD = 10240
NHALF = N_PAD // 2
XTRASH = 128
XTOT = NHALF + XTRASH
MROWS = M_PAD // NTILES
XZROWS = XTOT // NTILES
XCROWS = NHALF // NTILES
RBLK = 1000
MBLK = 640


def _k1a_body(x_ref, w_ref, b_ref, h_ref, s_ref):
    h = jnp.dot(x_ref[...], w_ref[...], preferred_element_type=jnp.float32)
    h = h + b_ref[...]
    h_ref[...] = h

    @pl.when(pl.program_id(0) == 0)
    def _():
        s_ref[...] = jnp.zeros_like(s_ref)

    s_ref[0:1, :] += jnp.sum(h, axis=0, keepdims=True)
    s_ref[1:2, :] += jnp.sum(h * h, axis=0, keepdims=True)


def _k1b_body(h_ref, s_ref, g_ref, be_ref, o_ref):
    mu = s_ref[0:1, :] / N
    var = s_ref[1:2, :] / N - mu * mu
    hn = (h_ref[...] - mu) * (lax.rsqrt(var + 1e-5) * g_ref[...]) + be_ref[...]
    o_ref[0] = hn[:, :128]
    o_ref[1] = hn[:, 128:]


def _stage1(X, W, b, gamma, beta):
    nb = N // RBLK
    H, S = pl.pallas_call(
        _k1a_body,
        grid=(nb,),
        in_specs=[
            pl.BlockSpec((RBLK, D), lambda i: (i, 0)),
            pl.BlockSpec((D, D), lambda i: (0, 0)),
            pl.BlockSpec((1, D), lambda i: (0, 0)),
        ],
        out_specs=[
            pl.BlockSpec((RBLK, D), lambda i: (i, 0)),
            pl.BlockSpec((8, D), lambda i: (0, 0)),
        ],
        out_shape=[
            jax.ShapeDtypeStruct((N, D), jnp.float32),
            jax.ShapeDtypeStruct((8, D), jnp.float32),
        ],
    )(X, W, b.reshape(1, D))
    H2 = pl.pallas_call(
        _k1b_body,
        grid=(nb,),
        in_specs=[
            pl.BlockSpec((RBLK, D), lambda i: (i, 0)),
            pl.BlockSpec((8, D), lambda i: (0, 0)),
            pl.BlockSpec((1, D), lambda i: (0, 0)),
            pl.BlockSpec((1, D), lambda i: (0, 0)),
        ],
        out_specs=pl.BlockSpec((2, RBLK, 128), lambda i: (0, i, 0)),
        out_shape=jax.ShapeDtypeStruct((2, N, 128), jnp.float32),
    )(H, S, gamma.reshape(1, D), beta.reshape(1, D))
    return H2


def _sc_v2e_body(h2, vA, eA, acc_out, vidx_v, eidx_v, rows_v, zbuf_v,
                 acc_sh, sem):
    c = lax.axis_index("c")
    s = lax.axis_index("s")
    cN = c * N

    def _zero_row(j, _):
        for k in range(8):
            zbuf_v[j, k * 16:(k + 1) * 16] = jnp.zeros((16,), jnp.float32)
        return 0
    lax.fori_loop(0, MROWS, _zero_row, 0)

    for g in range(3):
        pltpu.sync_copy(zbuf_v, acc_sh.at[pl.ds(s * MROWS, MROWS)])
        plsc.subcore_barrier()

        pltpu.sync_copy(vA.at[g, pl.ds(s * CHUNKS, CHUNKS)], vidx_v)
        pltpu.sync_copy(eA.at[g, pl.ds(s * CHUNKS, CHUNKS)], eidx_v)

        def _off_row(j, _):
            for k in range(8):
                sl = pl.ds(k * 16, 16)
                vidx_v[j, sl] = vidx_v[j, sl] + cN
            return 0
        lax.fori_loop(0, CHUNKS, _off_row, 0)

        def _chunk(j, _):
            pltpu.async_copy(h2.at[vidx_v.at[j]], rows_v, sem).wait()
            pltpu.sync_copy(rows_v, acc_sh.at[eidx_v.at[j]], add=True)
            return 0
        lax.fori_loop(0, CHUNKS, _chunk, 0)
        plsc.subcore_barrier()

        pltpu.sync_copy(acc_sh.at[pl.ds(s * MROWS, MROWS)],
                        acc_out.at[c, g, pl.ds(s * MROWS, MROWS)])


def _make_sc_v2e():
    mesh = plsc.VectorSubcoreMesh(core_axis_name="c", subcore_axis_name="s")
    return pl.kernel(
        _sc_v2e_body,
        mesh=mesh,
        out_type=jax.ShapeDtypeStruct((2, 3, M_PAD, 128), jnp.float32),
        scratch_types=[
            pltpu.VMEM((CHUNKS, 128), jnp.int32),
            pltpu.VMEM((CHUNKS, 128), jnp.int32),
            pltpu.VMEM((128, 128), jnp.float32),
            pltpu.VMEM((MROWS, 128), jnp.float32),
            pltpu.VMEM_SHARED((M_PAD, 128), jnp.float32),
            pltpu.SemaphoreType.DMA,
        ],
    )


HROWS = M_PAD * 16 // 128


def _sc_cnt_body(eA, cnt_out, eidx_v, hist_v, sem):
    c = lax.axis_index("c")
    s = lax.axis_index("s")
    del sem
    lanes = lax.iota(jnp.int32, 16)
    ones16 = jnp.ones((16,), jnp.float32)

    for g in range(3):
        def _zero(j, _):
            for k in range(8):
                hist_v[j, k * 16:(k + 1) * 16] = jnp.zeros((16,), jnp.float32)
            return 0
        lax.fori_loop(0, HROWS, _zero, 0)

        pltpu.sync_copy(
            eA.at[g, pl.ds((c * NTILES + s) * CCHUNKS, CCHUNKS)], eidx_v)

        def _chunk(j, _):
            for k in range(8):
                ev = eidx_v[j, k * 16:(k + 1) * 16]
                flat = ev * 16 + lanes
                plsc.addupdate_scatter(
                    hist_v, [lax.shift_right_logical(flat, 7), flat & 127],
                    ones16)
            return 0
        lax.fori_loop(0, CCHUNKS, _chunk, 0)

        pltpu.sync_copy(hist_v, cnt_out.at[c, s, g])


def _make_sc_cnt():
    mesh = plsc.VectorSubcoreMesh(core_axis_name="c", subcore_axis_name="s")
    return pl.kernel(
        _sc_cnt_body,
        mesh=mesh,
        compiler_params=pltpu.CompilerParams(needs_layout_passes=False),
        out_type=jax.ShapeDtypeStruct((2, NTILES, 3, HROWS, 128), jnp.float32),
        scratch_types=[
            pltpu.VMEM((CCHUNKS, 128), jnp.int32),
            pltpu.VMEM((HROWS, 128), jnp.float32),
            pltpu.SemaphoreType.DMA,
        ],
    )


def _kb_body(acc_ref, cnt_ref, w_ref, a_ref, z_ref):
    ch = jnp.sum(cnt_ref[...], axis=(0, 1, 2))
    ri = lax.broadcasted_iota(jnp.int32, (MBLK, MBLK // 8), 0)
    ci = lax.broadcasted_iota(jnp.int32, (MBLK, MBLK // 8), 1)
    U = (ri // 8 == ci).astype(jnp.float32)
    chr_ = jnp.dot(U, ch, preferred_element_type=jnp.float32)
    li = lax.broadcasted_iota(jnp.int32, (MBLK, 128), 1)
    ii = lax.broadcasted_iota(jnp.int32, (MBLK, 128), 0)
    sel = (li // 16) == (ii % 8)
    cnt = jnp.sum(jnp.where(sel, chr_, 0.0), axis=1, keepdims=True)
    cnt = jnp.maximum(cnt, 1.0)
    y0 = acc_ref[0, 0] / cnt
    y1 = acc_ref[1, 0] / cnt
    alpha = (jnp.dot(y0, w_ref[0], preferred_element_type=jnp.float32)
             + jnp.dot(y1, w_ref[1], preferred_element_type=jnp.float32))
    t = jnp.where(alpha >= 0, alpha, 0.2 * alpha)
    t = jnp.clip(t, 0.0, 5.0) * a_ref[pl.program_id(0), 0]
    z_ref[0, 0] = y0 * t
    z_ref[1, 0] = y1 * t


def _stage3(Acc, Cnt, w_e, avec):
    mb = M_PAD // MBLK
    return pl.pallas_call(
        _kb_body,
        grid=(3, mb),
        in_specs=[
            pl.BlockSpec((2, 1, MBLK, 128), lambda g, m: (0, g, m, 0)),
            pl.BlockSpec((2, NTILES, 1, MBLK * 16 // 128, 128),
                         lambda g, m: (0, 0, g, m, 0)),
            pl.BlockSpec((2, 128, 1), lambda g, m: (0, 0, 0)),
            pl.BlockSpec(memory_space=pltpu.SMEM),
        ],
        out_specs=pl.BlockSpec((2, 1, MBLK, 128), lambda g, m: (0, g, m, 0)),
        out_shape=jax.ShapeDtypeStruct((2, 3, M_PAD, 128), jnp.float32),
    )(Acc, Cnt, w_e.reshape(2, 128, 1), avec)


def _sc_e2v_body(zflat, vC, eC, x_out, vidx_v, eidx_v, rows_v, zbuf_v,
                 xacc_sh, sem):
    c = lax.axis_index("c")
    s = lax.axis_index("s")

    def _zero_row(j, _):
        for k in range(8):
            zbuf_v[j, k * 16:(k + 1) * 16] = jnp.zeros((16,), jnp.float32)
        return 0
    lax.fori_loop(0, XZROWS, _zero_row, 0)

    for p in range(2):
        pltpu.sync_copy(zbuf_v, xacc_sh.at[pl.ds(s * XZROWS, XZROWS)])
        plsc.subcore_barrier()

        for g in range(3):
            plane = (c * 3 + g) * M_PAD
            pltpu.sync_copy(vC.at[p, g, pl.ds(s * CHUNKS, CHUNKS)], vidx_v)
            pltpu.sync_copy(eC.at[g, pl.ds(s * CHUNKS, CHUNKS)], eidx_v)

            def _off_row(j, _):
                for k in range(8):
                    sl = pl.ds(k * 16, 16)
                    eidx_v[j, sl] = eidx_v[j, sl] + plane
                return 0
            lax.fori_loop(0, CHUNKS, _off_row, 0)

            def _chunk(j, _):
                pltpu.async_copy(zflat.at[eidx_v.at[j]], rows_v, sem).wait()
                pltpu.sync_copy(rows_v, xacc_sh.at[vidx_v.at[j]], add=True)
                return 0
            lax.fori_loop(0, CHUNKS, _chunk, 0)
        plsc.subcore_barrier()

        pltpu.sync_copy(xacc_sh.at[pl.ds(s * XCROWS, XCROWS)],
                        x_out.at[c, p, pl.ds(s * XCROWS, XCROWS)])
        plsc.subcore_barrier()


def _make_sc_e2v():
    mesh = plsc.VectorSubcoreMesh(core_axis_name="c", subcore_axis_name="s")
    return pl.kernel(
        _sc_e2v_body,
        mesh=mesh,
        out_type=jax.ShapeDtypeStruct((2, 2, NHALF, 128), jnp.float32),
        scratch_types=[
            pltpu.VMEM((CHUNKS, 128), jnp.int32),
            pltpu.VMEM((CHUNKS, 128), jnp.int32),
            pltpu.VMEM((128, 128), jnp.float32),
            pltpu.VMEM((XZROWS, 128), jnp.float32),
            pltpu.VMEM_SHARED((XTOT, 128), jnp.float32),
            pltpu.SemaphoreType.DMA,
        ],
    )


def _kd_body(x_ref, o_ref):
    x0 = x_ref[0]
    x1 = x_ref[1]
    o_ref[:, :128] = jnp.where(x0 >= 0, x0, 0.01 * x0)
    o_ref[:, 128:] = jnp.where(x1 >= 0, x1, 0.01 * x1)


def _stage5(Xout):
    nb = N // RBLK
    return pl.pallas_call(
        _kd_body,
        grid=(nb,),
        in_specs=[pl.BlockSpec((2, RBLK, 128), lambda i: (0, i, 0))],
        out_specs=pl.BlockSpec((RBLK, D), lambda i: (i, 0)),
        out_shape=jax.ShapeDtypeStruct((N, D), jnp.float32),
    )(Xout)


def _pad_idx(v, e):
    ar = jnp.arange(E_PAD - E, dtype=jnp.int32)
    vA = jnp.concatenate([v, ar % 64])
    eA = jnp.concatenate([e, M + ar % (M_PAD - M)])
    vfull = jnp.concatenate([v, N + ar % (N_PAD - N)])
    spread = jnp.arange(E_PAD, dtype=jnp.int32) % XTRASH + NHALF
    vC = []
    for p in range(2):
        inp = (vfull >= p * NHALF) & (vfull < (p + 1) * NHALF)
        vC.append(jnp.where(inp, vfull - p * NHALF, spread))
    return vA, eA, jnp.stack(vC)


def kernel(X, v_hier, e_hier, v_cooc, e_cooc, v_cite, e_cite,
           W, b, gamma, beta, w_e, a1, a2, a3):
    groups = [(v_hier, e_hier), (v_cooc, e_cooc), (v_cite, e_cite)]
    vAs, eAs, vCs = [], [], []
    for v, e in groups:
        vA, eA, vC = _pad_idx(v, e)
        vAs.append(vA)
        eAs.append(eA)
        vCs.append(vC)
    vA3 = jnp.stack(vAs).reshape(3, E_PAD // 128, 128)
    eA3 = jnp.stack(eAs).reshape(3, E_PAD // 128, 128)
    vC3 = jnp.stack(vCs, axis=1).reshape(2, 3, E_PAD // 128, 128)

    H2 = _stage1(X, W, b, gamma, beta).reshape(2 * N, 128)
    Acc = _make_sc_v2e()(H2, vA3, eA3)
    Cnt = _make_sc_cnt()(eA3)
    avec = jnp.concatenate([a1.ravel(), a2.ravel(), a3.ravel()]).reshape(3, 1)
    Z = _stage3(Acc, Cnt, w_e, avec)
    Xout = _make_sc_e2v()(Z.reshape(6 * M_PAD, 128), vC3, eA3)
    Xo = _stage5(Xout.reshape(2, N_PAD, 128))
    a = jnp.concatenate([a1.ravel(), a2.ravel(), a3.ravel()])
    return (Xo, a)


# speedup vs baseline: 5.5394x; 1.2098x over previous
import jax
import jax.numpy as jnp
from jax import lax
from jax.experimental import pallas as pl
from jax.experimental.pallas import tpu as pltpu
from jax.experimental.pallas import tpu_sc as plsc

N = 10000
D = 256
M = 5000
E = 160000

NTILES = 16
E_PAD = 163840
CHUNKS = E_PAD // NTILES // 128
CCHUNKS = CHUNKS // 2
M_PAD = 5120
N_PAD = 10240
NHALF = N_PAD // 2
XTRASH = 128
XTOT = NHALF + XTRASH
MROWS = M_PAD // NTILES
XZROWS = XTOT // NTILES
XCROWS = NHALF // NTILES
RBLK = 1000
MBLK = 640


def _k1a_body(x_ref, w_ref, b_ref, h_ref, s_ref):
    h = jnp.dot(x_ref[...], w_ref[...], preferred_element_type=jnp.float32)
    h = h + b_ref[...]
    h_ref[...] = h

    @pl.when(pl.program_id(0) == 0)
    def _():
        s_ref[...] = jnp.zeros_like(s_ref)

    s_ref[0:1, :] += jnp.sum(h, axis=0, keepdims=True)
    s_ref[1:2, :] += jnp.sum(h * h, axis=0, keepdims=True)


def _k1b_body(h_ref, s_ref, g_ref, be_ref, o_ref):
    mu = s_ref[0:1, :] / N
    var = s_ref[1:2, :] / N - mu * mu
    hn = (h_ref[...] - mu) * (lax.rsqrt(var + 1e-5) * g_ref[...]) + be_ref[...]
    o_ref[0] = hn[:, :128]
    o_ref[1] = hn[:, 128:]


def _stage1(X, W, b, gamma, beta):
    nb = N // RBLK
    H, S = pl.pallas_call(
        _k1a_body,
        grid=(nb,),
        in_specs=[
            pl.BlockSpec((RBLK, D), lambda i: (i, 0)),
            pl.BlockSpec((D, D), lambda i: (0, 0)),
            pl.BlockSpec((1, D), lambda i: (0, 0)),
        ],
        out_specs=[
            pl.BlockSpec((RBLK, D), lambda i: (i, 0)),
            pl.BlockSpec((8, D), lambda i: (0, 0)),
        ],
        out_shape=[
            jax.ShapeDtypeStruct((N, D), jnp.float32),
            jax.ShapeDtypeStruct((8, D), jnp.float32),
        ],
    )(X, W, b.reshape(1, D))
    H2 = pl.pallas_call(
        _k1b_body,
        grid=(nb,),
        in_specs=[
            pl.BlockSpec((RBLK, D), lambda i: (i, 0)),
            pl.BlockSpec((8, D), lambda i: (0, 0)),
            pl.BlockSpec((1, D), lambda i: (0, 0)),
            pl.BlockSpec((1, D), lambda i: (0, 0)),
        ],
        out_specs=pl.BlockSpec((2, RBLK, 128), lambda i: (0, i, 0)),
        out_shape=jax.ShapeDtypeStruct((2, N, 128), jnp.float32),
    )(H, S, gamma.reshape(1, D), beta.reshape(1, D))
    return H2


def _sc_v2e_body(h2, vA, eA, acc_out, vidx_v, eidx_v, rows_v, zbuf_v,
                 acc_sh, sem):
    c = lax.axis_index("c")
    s = lax.axis_index("s")
    cN = c * N

    def _zero_row(j, _):
        for k in range(8):
            zbuf_v[j, k * 16:(k + 1) * 16] = jnp.zeros((16,), jnp.float32)
        return 0
    lax.fori_loop(0, MROWS, _zero_row, 0)

    for g in range(3):
        pltpu.sync_copy(zbuf_v, acc_sh.at[pl.ds(s * MROWS, MROWS)])
        plsc.subcore_barrier()

        pltpu.sync_copy(vA.at[g, pl.ds(s * CHUNKS, CHUNKS)], vidx_v)
        pltpu.sync_copy(eA.at[g, pl.ds(s * CHUNKS, CHUNKS)], eidx_v)

        def _off_row(j, _):
            for k in range(8):
                sl = pl.ds(k * 16, 16)
                vidx_v[j, sl] = vidx_v[j, sl] + cN
            return 0
        lax.fori_loop(0, CHUNKS, _off_row, 0)

        def _chunk(j, _):
            pltpu.async_copy(h2.at[vidx_v.at[j]], rows_v, sem).wait()
            pltpu.sync_copy(rows_v, acc_sh.at[eidx_v.at[j]], add=True)
            return 0
        lax.fori_loop(0, CHUNKS, _chunk, 0)
        plsc.subcore_barrier()

        pltpu.sync_copy(acc_sh.at[pl.ds(s * MROWS, MROWS)],
                        acc_out.at[c, g, pl.ds(s * MROWS, MROWS)])


def _make_sc_v2e():
    mesh = plsc.VectorSubcoreMesh(core_axis_name="c", subcore_axis_name="s")
    return pl.kernel(
        _sc_v2e_body,
        mesh=mesh,
        out_type=jax.ShapeDtypeStruct((2, 3, M_PAD, 128), jnp.float32),
        scratch_types=[
            pltpu.VMEM((CHUNKS, 128), jnp.int32),
            pltpu.VMEM((CHUNKS, 128), jnp.int32),
            pltpu.VMEM((128, 128), jnp.float32),
            pltpu.VMEM((MROWS, 128), jnp.float32),
            pltpu.VMEM_SHARED((M_PAD, 128), jnp.float32),
            pltpu.SemaphoreType.DMA,
        ],
    )


HROWS = M_PAD * 16 // 128


def _sc_cnt_body(eA, cnt_out, eidx_v, hist_v, sem):
    c = lax.axis_index("c")
    s = lax.axis_index("s")
    del sem
    lanes = lax.iota(jnp.int32, 16)
    ones16 = jnp.ones((16,), jnp.float32)

    for g in range(3):
        def _zero(j, _):
            for k in range(8):
                hist_v[j, k * 16:(k + 1) * 16] = jnp.zeros((16,), jnp.float32)
            return 0
        lax.fori_loop(0, HROWS, _zero, 0)

        pltpu.sync_copy(
            eA.at[g, pl.ds((c * NTILES + s) * CCHUNKS, CCHUNKS)], eidx_v)

        def _chunk(j, _):
            for k in range(8):
                ev = eidx_v[j, k * 16:(k + 1) * 16]
                flat = ev * 16 + lanes
                plsc.addupdate_scatter(
                    hist_v, [lax.shift_right_logical(flat, 7), flat & 127],
                    ones16)
            return 0
        lax.fori_loop(0, CCHUNKS, _chunk, 0)

        pltpu.sync_copy(hist_v, cnt_out.at[c, s, g])


def _make_sc_cnt():
    mesh = plsc.VectorSubcoreMesh(core_axis_name="c", subcore_axis_name="s")
    return pl.kernel(
        _sc_cnt_body,
        mesh=mesh,
        compiler_params=pltpu.CompilerParams(needs_layout_passes=False),
        out_type=jax.ShapeDtypeStruct((2, NTILES, 3, HROWS, 128), jnp.float32),
        scratch_types=[
            pltpu.VMEM((CCHUNKS, 128), jnp.int32),
            pltpu.VMEM((HROWS, 128), jnp.float32),
            pltpu.SemaphoreType.DMA,
        ],
    )


def _kb_body(acc_ref, cnt_ref, w_ref, a_ref, z_ref):
    ch = jnp.sum(cnt_ref[...], axis=(0, 1, 2))
    ri = lax.broadcasted_iota(jnp.int32, (MBLK, MBLK // 8), 0)
    ci = lax.broadcasted_iota(jnp.int32, (MBLK, MBLK // 8), 1)
    U = (ri // 8 == ci).astype(jnp.float32)
    chr_ = jnp.dot(U, ch, preferred_element_type=jnp.float32)
    li = lax.broadcasted_iota(jnp.int32, (MBLK, 128), 1)
    ii = lax.broadcasted_iota(jnp.int32, (MBLK, 128), 0)
    sel = (li // 16) == (ii % 8)
    cnt = jnp.sum(jnp.where(sel, chr_, 0.0), axis=1, keepdims=True)
    cnt = jnp.maximum(cnt, 1.0)
    y0 = acc_ref[0, 0] / cnt
    y1 = acc_ref[1, 0] / cnt
    alpha = (jnp.dot(y0, w_ref[0], preferred_element_type=jnp.float32)
             + jnp.dot(y1, w_ref[1], preferred_element_type=jnp.float32))
    t = jnp.where(alpha >= 0, alpha, 0.2 * alpha)
    t = jnp.clip(t, 0.0, 5.0) * a_ref[pl.program_id(0), 0]
    z0 = y0 * t
    z1 = y1 * t
    z_ref[0, 0, 0] = z0[:, :64]
    z_ref[0, 1, 0] = z0[:, 64:]
    z_ref[1, 0, 0] = z1[:, :64]
    z_ref[1, 1, 0] = z1[:, 64:]


def _stage3(Acc, Cnt, w_e, avec):
    mb = M_PAD // MBLK
    return pl.pallas_call(
        _kb_body,
        grid=(3, mb),
        in_specs=[
            pl.BlockSpec((2, 1, MBLK, 128), lambda g, m: (0, g, m, 0)),
            pl.BlockSpec((2, NTILES, 1, MBLK * 16 // 128, 128),
                         lambda g, m: (0, 0, g, m, 0)),
            pl.BlockSpec((2, 128, 1), lambda g, m: (0, 0, 0)),
            pl.BlockSpec(memory_space=pltpu.SMEM),
        ],
        out_specs=pl.BlockSpec((2, 2, 1, MBLK, 64), lambda g, m: (0, 0, g, m, 0)),
        out_shape=jax.ShapeDtypeStruct((2, 2, 3, M_PAD, 64), jnp.float32),
    )(Acc, Cnt, w_e.reshape(2, 128, 1), avec)


def _sc_e2v_body(zflat, vC, eC, x_out, vidx_v, eidx_v, rows_v, zbuf_v,
                 xacc_sh, sem):
    c = lax.axis_index("c")
    s = lax.axis_index("s")
    XROWS = N_PAD // NTILES

    def _zero_row(j, _):
        for k in range(4):
            zbuf_v[j, k * 16:(k + 1) * 16] = jnp.zeros((16,), jnp.float32)
        return 0
    lax.fori_loop(0, XROWS, _zero_row, 0)

    for p in range(2):
        pltpu.sync_copy(zbuf_v, xacc_sh.at[pl.ds(s * XROWS, XROWS)])
        plsc.subcore_barrier()

        for g in range(3):
            plane = ((c * 2 + p) * 3 + g) * M_PAD
            pltpu.sync_copy(vC.at[g, pl.ds(s * CHUNKS, CHUNKS)], vidx_v)
            pltpu.sync_copy(eC.at[g, pl.ds(s * CHUNKS, CHUNKS)], eidx_v)

            def _off_row(j, _):
                for k in range(8):
                    sl = pl.ds(k * 16, 16)
                    eidx_v[j, sl] = eidx_v[j, sl] + plane
                return 0
            lax.fori_loop(0, CHUNKS, _off_row, 0)

            def _chunk(j, _):
                pltpu.async_copy(zflat.at[eidx_v.at[j]], rows_v, sem).wait()
                pltpu.sync_copy(rows_v, xacc_sh.at[vidx_v.at[j]], add=True)
                return 0
            lax.fori_loop(0, CHUNKS, _chunk, 0)
        plsc.subcore_barrier()

        pltpu.sync_copy(xacc_sh.at[pl.ds(s * XROWS, XROWS)],
                        x_out.at[c, p, pl.ds(s * XROWS, XROWS)])
        plsc.subcore_barrier()


def _make_sc_e2v():
    mesh = plsc.VectorSubcoreMesh(core_axis_name="c", subcore_axis_name="s")
    return pl.kernel(
        _sc_e2v_body,
        mesh=mesh,
        compiler_params=pltpu.CompilerParams(use_tc_tiling_on_sc=False),
        out_type=jax.ShapeDtypeStruct((2, 2, N_PAD, 64), jnp.float32),
        scratch_types=[
            pltpu.VMEM((CHUNKS, 128), jnp.int32),
            pltpu.VMEM((CHUNKS, 128), jnp.int32),
            pltpu.VMEM((128, 64), jnp.float32),
            pltpu.VMEM((N_PAD // NTILES, 64), jnp.float32),
            pltpu.VMEM_SHARED((N_PAD, 64), jnp.float32),
            pltpu.SemaphoreType.DMA,
        ],
    )


def _kd_body(x_ref, o_ref):
    for c in range(2):
        for p in range(2):
            xq = x_ref[c, p]
            lo = (c * 2 + p) * 64
            o_ref[:, lo:lo + 64] = jnp.where(xq >= 0, xq, 0.01 * xq)


def _stage5(Xout):
    nb = N // RBLK
    return pl.pallas_call(
        _kd_body,
        grid=(nb,),
        in_specs=[pl.BlockSpec((2, 2, RBLK, 64), lambda i: (0, 0, i, 0))],
        out_specs=pl.BlockSpec((RBLK, D), lambda i: (i, 0)),
        out_shape=jax.ShapeDtypeStruct((N, D), jnp.float32),
    )(Xout)


def _pad_idx(v, e):
    ar = jnp.arange(E_PAD - E, dtype=jnp.int32)
    vA = jnp.concatenate([v, ar % 64])
    eA = jnp.concatenate([e, M + ar % (M_PAD - M)])
    vC = jnp.concatenate([v, N + ar % (N_PAD - N)])
    return vA, eA, vC


def kernel(X, v_hier, e_hier, v_cooc, e_cooc, v_cite, e_cite,
           W, b, gamma, beta, w_e, a1, a2, a3):
    groups = [(v_hier, e_hier), (v_cooc, e_cooc), (v_cite, e_cite)]
    vAs, eAs, vCs = [], [], []
    for v, e in groups:
        vA, eA, vC = _pad_idx(v, e)
        vAs.append(vA)
        eAs.append(eA)
        vCs.append(vC)
    vA3 = jnp.stack(vAs).reshape(3, E_PAD // 128, 128)
    eA3 = jnp.stack(eAs).reshape(3, E_PAD // 128, 128)
    vC3 = jnp.stack(vCs).reshape(3, E_PAD // 128, 128)

    H2 = _stage1(X, W, b, gamma, beta).reshape(2 * N, 128)
    Acc = _make_sc_v2e()(H2, vA3, eA3)
    Cnt = _make_sc_cnt()(eA3)
    avec = jnp.concatenate([a1.ravel(), a2.ravel(), a3.ravel()]).reshape(3, 1)
    Z = _stage3(Acc, Cnt, w_e, avec)
    Xout = _make_sc_e2v()(Z.reshape(12 * M_PAD, 64), vC3, eA3)
    Xo = _stage5(Xout)
    a = jnp.concatenate([a1.ravel(), a2.ravel(), a3.ravel()])
    return (Xo, a)


# speedup vs baseline: 7.0911x; 1.2801x over previous
import jax
import jax.numpy as jnp
from jax import lax
from jax.experimental import pallas as pl
from jax.experimental.pallas import tpu as pltpu
from jax.experimental.pallas import tpu_sc as plsc

N = 10000
D = 256
M = 5000
E = 160000

NTILES = 16
E_PAD = 163840
CHUNKS = E_PAD // NTILES // 128
CCHUNKS = CHUNKS // 2
M_PAD = 5120
N_PAD = 10240
NHALF = N_PAD // 2
XTRASH = 128
XTOT = NHALF + XTRASH
MROWS = M_PAD // NTILES
XZROWS = XTOT // NTILES
XCROWS = NHALF // NTILES
RBLK = 1000
MBLK = 640


def _k1a_body(x_ref, w_ref, b_ref, h_ref, s_ref):
    h = jnp.dot(x_ref[...], w_ref[...], preferred_element_type=jnp.float32)
    h = h + b_ref[...]
    h_ref[...] = h

    @pl.when(pl.program_id(0) == 0)
    def _():
        s_ref[...] = jnp.zeros_like(s_ref)

    s_ref[0:1, :] += jnp.sum(h, axis=0, keepdims=True)
    s_ref[1:2, :] += jnp.sum(h * h, axis=0, keepdims=True)


def _k1b_body(h_ref, s_ref, g_ref, be_ref, o_ref):
    mu = s_ref[0:1, :] / N
    var = s_ref[1:2, :] / N - mu * mu
    hn = (h_ref[...] - mu) * (lax.rsqrt(var + 1e-5) * g_ref[...]) + be_ref[...]
    o_ref[0] = hn[:, :128]
    o_ref[1] = hn[:, 128:]


def _stage1(X, W, b, gamma, beta):
    nb = N // RBLK
    H, S = pl.pallas_call(
        _k1a_body,
        grid=(nb,),
        in_specs=[
            pl.BlockSpec((RBLK, D), lambda i: (i, 0)),
            pl.BlockSpec((D, D), lambda i: (0, 0)),
            pl.BlockSpec((1, D), lambda i: (0, 0)),
        ],
        out_specs=[
            pl.BlockSpec((RBLK, D), lambda i: (i, 0)),
            pl.BlockSpec((8, D), lambda i: (0, 0)),
        ],
        out_shape=[
            jax.ShapeDtypeStruct((N, D), jnp.float32),
            jax.ShapeDtypeStruct((8, D), jnp.float32),
        ],
    )(X, W, b.reshape(1, D))
    H2 = pl.pallas_call(
        _k1b_body,
        grid=(nb,),
        in_specs=[
            pl.BlockSpec((RBLK, D), lambda i: (i, 0)),
            pl.BlockSpec((8, D), lambda i: (0, 0)),
            pl.BlockSpec((1, D), lambda i: (0, 0)),
            pl.BlockSpec((1, D), lambda i: (0, 0)),
        ],
        out_specs=pl.BlockSpec((2, RBLK, 128), lambda i: (0, i, 0)),
        out_shape=jax.ShapeDtypeStruct((2, N, 128), jnp.float32),
    )(H, S, gamma.reshape(1, D), beta.reshape(1, D))
    return H2


def _gs_pipeline(src, idx_v, out_idx_v, dst_sh, rows_a, rows_b, sem_a, sem_b):
    def _wait(rows, sem):
        pltpu.make_async_copy(src.at[idx_v.at[0]], rows, sem).wait()

    pltpu.async_copy(src.at[idx_v.at[0]], rows_a, sem_a)

    def _pair(i, _):
        ja = 2 * i
        pltpu.async_copy(src.at[idx_v.at[ja + 1]], rows_b, sem_b)
        _wait(rows_a, sem_a)
        pltpu.sync_copy(rows_a, dst_sh.at[out_idx_v.at[ja]], add=True)
        pltpu.async_copy(src.at[idx_v.at[(ja + 2) % CHUNKS]], rows_a, sem_a)
        _wait(rows_b, sem_b)
        pltpu.sync_copy(rows_b, dst_sh.at[out_idx_v.at[ja + 1]], add=True)
        return 0

    lax.fori_loop(0, CHUNKS // 2, _pair, 0)
    _wait(rows_a, sem_a)


def _sc_v2e_body(h2, vA, eA, acc_out, vidx_v, eidx_v, rows_a, rows_b,
                 zbuf_v, acc_sh, sem_a, sem_b):
    c = lax.axis_index("c")
    s = lax.axis_index("s")
    cN = c * N

    def _zero_row(j, _):
        for k in range(8):
            zbuf_v[j, k * 16:(k + 1) * 16] = jnp.zeros((16,), jnp.float32)
        return 0
    lax.fori_loop(0, MROWS, _zero_row, 0)

    for g in range(3):
        pltpu.sync_copy(zbuf_v, acc_sh.at[pl.ds(s * MROWS, MROWS)])
        plsc.subcore_barrier()

        pltpu.sync_copy(vA.at[g, pl.ds(s * CHUNKS, CHUNKS)], vidx_v)
        pltpu.sync_copy(eA.at[g, pl.ds(s * CHUNKS, CHUNKS)], eidx_v)

        def _off_row(j, _):
            for k in range(8):
                sl = pl.ds(k * 16, 16)
                vidx_v[j, sl] = vidx_v[j, sl] + cN
            return 0
        lax.fori_loop(0, CHUNKS, _off_row, 0)

        def _chunk(j, _):
            pltpu.async_copy(h2.at[vidx_v.at[j]], rows_a, sem_a).wait()
            pltpu.sync_copy(rows_a, acc_sh.at[eidx_v.at[j]], add=True)
            return 0
        lax.fori_loop(0, CHUNKS, _chunk, 0)
        plsc.subcore_barrier()

        pltpu.sync_copy(acc_sh.at[pl.ds(s * MROWS, MROWS)],
                        acc_out.at[c, g, pl.ds(s * MROWS, MROWS)])


def _make_sc_v2e():
    mesh = plsc.VectorSubcoreMesh(core_axis_name="c", subcore_axis_name="s")
    return pl.kernel(
        _sc_v2e_body,
        mesh=mesh,
        out_type=jax.ShapeDtypeStruct((2, 3, M_PAD, 128), jnp.float32),
        scratch_types=[
            pltpu.VMEM((CHUNKS, 128), jnp.int32),
            pltpu.VMEM((CHUNKS, 128), jnp.int32),
            pltpu.VMEM((128, 128), jnp.float32),
            pltpu.VMEM((128, 128), jnp.float32),
            pltpu.VMEM((MROWS, 128), jnp.float32),
            pltpu.VMEM_SHARED((M_PAD, 128), jnp.float32),
            pltpu.SemaphoreType.DMA,
            pltpu.SemaphoreType.DMA,
        ],
    )


HROWS = M_PAD * 16 // 128


def _sc_cnt_body(eA, cnt_out, eidx_v, hist_v, sem):
    c = lax.axis_index("c")
    s = lax.axis_index("s")
    del sem
    lanes = lax.iota(jnp.int32, 16)
    ones16 = jnp.ones((16,), jnp.float32)

    for g in range(3):
        def _zero(j, _):
            for k in range(8):
                hist_v[j, k * 16:(k + 1) * 16] = jnp.zeros((16,), jnp.float32)
            return 0
        lax.fori_loop(0, HROWS, _zero, 0)

        pltpu.sync_copy(
            eA.at[g, pl.ds((c * NTILES + s) * CCHUNKS, CCHUNKS)], eidx_v)

        def _chunk(j, _):
            for k in range(8):
                ev = eidx_v[j, k * 16:(k + 1) * 16]
                flat = ev * 16 + lanes
                plsc.addupdate_scatter(
                    hist_v, [lax.shift_right_logical(flat, 7), flat & 127],
                    ones16)
            return 0
        lax.fori_loop(0, CCHUNKS, _chunk, 0)

        pltpu.sync_copy(hist_v, cnt_out.at[c, s, g])


def _make_sc_cnt():
    mesh = plsc.VectorSubcoreMesh(core_axis_name="c", subcore_axis_name="s")
    return pl.kernel(
        _sc_cnt_body,
        mesh=mesh,
        compiler_params=pltpu.CompilerParams(needs_layout_passes=False),
        out_type=jax.ShapeDtypeStruct((2, NTILES, 3, HROWS, 128), jnp.float32),
        scratch_types=[
            pltpu.VMEM((CCHUNKS, 128), jnp.int32),
            pltpu.VMEM((HROWS, 128), jnp.float32),
            pltpu.SemaphoreType.DMA,
        ],
    )


def _kb_body(acc_ref, cnt_ref, w_ref, a_ref, z_ref):
    ch = jnp.sum(cnt_ref[...], axis=(0, 1, 2))
    ri = lax.broadcasted_iota(jnp.int32, (MBLK, MBLK // 8), 0)
    ci = lax.broadcasted_iota(jnp.int32, (MBLK, MBLK // 8), 1)
    U = (ri // 8 == ci).astype(jnp.float32)
    chr_ = jnp.dot(U, ch, preferred_element_type=jnp.float32)
    li = lax.broadcasted_iota(jnp.int32, (MBLK, 128), 1)
    ii = lax.broadcasted_iota(jnp.int32, (MBLK, 128), 0)
    sel = (li // 16) == (ii % 8)
    cnt = jnp.sum(jnp.where(sel, chr_, 0.0), axis=1, keepdims=True)
    cnt = jnp.maximum(cnt, 1.0)
    y0 = acc_ref[0, 0] / cnt
    y1 = acc_ref[1, 0] / cnt
    alpha = (jnp.dot(y0, w_ref[0], preferred_element_type=jnp.float32)
             + jnp.dot(y1, w_ref[1], preferred_element_type=jnp.float32))
    t = jnp.where(alpha >= 0, alpha, 0.2 * alpha)
    t = jnp.clip(t, 0.0, 5.0) * a_ref[pl.program_id(0), 0]
    z0 = y0 * t
    z1 = y1 * t
    z_ref[0, 0, 0] = z0[:, :64]
    z_ref[0, 1, 0] = z0[:, 64:]
    z_ref[1, 0, 0] = z1[:, :64]
    z_ref[1, 1, 0] = z1[:, 64:]


def _stage3(Acc, Cnt, w_e, avec):
    mb = M_PAD // MBLK
    return pl.pallas_call(
        _kb_body,
        grid=(3, mb),
        in_specs=[
            pl.BlockSpec((2, 1, MBLK, 128), lambda g, m: (0, g, m, 0)),
            pl.BlockSpec((2, NTILES, 1, MBLK * 16 // 128, 128),
                         lambda g, m: (0, 0, g, m, 0)),
            pl.BlockSpec((2, 128, 1), lambda g, m: (0, 0, 0)),
            pl.BlockSpec(memory_space=pltpu.SMEM),
        ],
        out_specs=pl.BlockSpec((2, 2, 1, MBLK, 64), lambda g, m: (0, 0, g, m, 0)),
        out_shape=jax.ShapeDtypeStruct((2, 2, 3, M_PAD, 64), jnp.float32),
    )(Acc, Cnt, w_e.reshape(2, 128, 1), avec)


def _sc_e2v_body(zflat, vC, eC, x_out, vidx_v, eidx_v, rows_a, rows_b,
                 zbuf_v, xacc_sh, sem_a, sem_b):
    c = lax.axis_index("c")
    s = lax.axis_index("s")
    XROWS = N_PAD // NTILES

    def _zero_row(j, _):
        for k in range(4):
            zbuf_v[j, k * 16:(k + 1) * 16] = jnp.zeros((16,), jnp.float32)
        return 0
    lax.fori_loop(0, XROWS, _zero_row, 0)

    for p in range(2):
        pltpu.sync_copy(zbuf_v, xacc_sh.at[pl.ds(s * XROWS, XROWS)])
        plsc.subcore_barrier()

        for g in range(3):
            plane = ((c * 2 + p) * 3 + g) * M_PAD
            pltpu.sync_copy(vC.at[g, pl.ds(s * CHUNKS, CHUNKS)], vidx_v)
            pltpu.sync_copy(eC.at[g, pl.ds(s * CHUNKS, CHUNKS)], eidx_v)

            def _off_row(j, _):
                for k in range(8):
                    sl = pl.ds(k * 16, 16)
                    eidx_v[j, sl] = eidx_v[j, sl] + plane
                return 0
            lax.fori_loop(0, CHUNKS, _off_row, 0)

            _gs_pipeline(zflat, eidx_v, vidx_v, xacc_sh, rows_a, rows_b,
                         sem_a, sem_b)
        plsc.subcore_barrier()

        pltpu.sync_copy(xacc_sh.at[pl.ds(s * XROWS, XROWS)],
                        x_out.at[c, p, pl.ds(s * XROWS, XROWS)])
        plsc.subcore_barrier()


def _make_sc_e2v():
    mesh = plsc.VectorSubcoreMesh(core_axis_name="c", subcore_axis_name="s")
    return pl.kernel(
        _sc_e2v_body,
        mesh=mesh,
        compiler_params=pltpu.CompilerParams(use_tc_tiling_on_sc=False),
        out_type=jax.ShapeDtypeStruct((2, 2, N_PAD, 64), jnp.float32),
        scratch_types=[
            pltpu.VMEM((CHUNKS, 128), jnp.int32),
            pltpu.VMEM((CHUNKS, 128), jnp.int32),
            pltpu.VMEM((128, 64), jnp.float32),
            pltpu.VMEM((128, 64), jnp.float32),
            pltpu.VMEM((N_PAD // NTILES, 64), jnp.float32),
            pltpu.VMEM_SHARED((N_PAD, 64), jnp.float32),
            pltpu.SemaphoreType.DMA,
            pltpu.SemaphoreType.DMA,
        ],
    )


def _kd_body(x_ref, o_ref):
    for c in range(2):
        for p in range(2):
            xq = x_ref[c, p]
            lo = (c * 2 + p) * 64
            o_ref[:, lo:lo + 64] = jnp.where(xq >= 0, xq, 0.01 * xq)


def _stage5(Xout):
    nb = N // RBLK
    return pl.pallas_call(
        _kd_body,
        grid=(nb,),
        in_specs=[pl.BlockSpec((2, 2, RBLK, 64), lambda i: (0, 0, i, 0))],
        out_specs=pl.BlockSpec((RBLK, D), lambda i: (i, 0)),
        out_shape=jax.ShapeDtypeStruct((N, D), jnp.float32),
    )(Xout)


def _pad_idx(v, e):
    ar = jnp.arange(E_PAD - E, dtype=jnp.int32)
    vA = jnp.concatenate([v, ar % 64])
    eA = jnp.concatenate([e, M + ar % (M_PAD - M)])
    vC = jnp.concatenate([v, N + ar % (N_PAD - N)])
    return vA, eA, vC


def kernel(X, v_hier, e_hier, v_cooc, e_cooc, v_cite, e_cite,
           W, b, gamma, beta, w_e, a1, a2, a3):
    groups = [(v_hier, e_hier), (v_cooc, e_cooc), (v_cite, e_cite)]
    vAs, eAs, vCs = [], [], []
    for v, e in groups:
        vA, eA, vC = _pad_idx(v, e)
        vAs.append(vA)
        eAs.append(eA)
        vCs.append(vC)
    vA3 = jnp.stack(vAs).reshape(3, E_PAD // 128, 128)
    eA3 = jnp.stack(eAs).reshape(3, E_PAD // 128, 128)
    vC3 = jnp.stack(vCs).reshape(3, E_PAD // 128, 128)

    H2 = _stage1(X, W, b, gamma, beta).reshape(2 * N, 128)
    Acc = _make_sc_v2e()(H2, vA3, eA3)
    Cnt = _make_sc_cnt()(eA3)
    avec = jnp.concatenate([a1.ravel(), a2.ravel(), a3.ravel()]).reshape(3, 1)
    Z = _stage3(Acc, Cnt, w_e, avec)
    Xout = _make_sc_e2v()(Z.reshape(12 * M_PAD, 64), vC3, eA3)
    Xo = _stage5(Xout)
    a = jnp.concatenate([a1.ravel(), a2.ravel(), a3.ravel()])
    return (Xo, a)


# speedup vs baseline: 7.1023x; 1.0016x over previous
import jax
import jax.numpy as jnp
from jax import lax
from jax.experimental import pallas as pl
from jax.experimental.pallas import tpu as pltpu
from jax.experimental.pallas import tpu_sc as plsc

N = 10000
D = 256
M = 5000
E = 160000

NTILES = 16
E_PAD = 163840
CHUNKS = E_PAD // NTILES // 128
CCHUNKS = CHUNKS // 2
M_PAD = 5120
N_PAD = 10240
NHALF = N_PAD // 2
XTRASH = 128
XTOT = NHALF + XTRASH
MROWS = M_PAD // NTILES
XZROWS = XTOT // NTILES
XCROWS = NHALF // NTILES
RBLK = 1000
MBLK = 640


def _k1a_body(x_ref, w_ref, b_ref, h_ref, s_ref):
    h = jnp.dot(x_ref[...], w_ref[...], preferred_element_type=jnp.float32)
    h = h + b_ref[...]
    h_ref[...] = h

    @pl.when(pl.program_id(0) == 0)
    def _():
        s_ref[...] = jnp.zeros_like(s_ref)

    s_ref[0:1, :] += jnp.sum(h, axis=0, keepdims=True)
    s_ref[1:2, :] += jnp.sum(h * h, axis=0, keepdims=True)


def _k1b_body(h_ref, s_ref, g_ref, be_ref, o_ref):
    mu = s_ref[0:1, :] / N
    var = s_ref[1:2, :] / N - mu * mu
    hn = (h_ref[...] - mu) * (lax.rsqrt(var + 1e-5) * g_ref[...]) + be_ref[...]
    o_ref[0] = hn[:, :128]
    o_ref[1] = hn[:, 128:]


def _stage1(X, W, b, gamma, beta):
    nb = N // RBLK
    H, S = pl.pallas_call(
        _k1a_body,
        grid=(nb,),
        in_specs=[
            pl.BlockSpec((RBLK, D), lambda i: (i, 0)),
            pl.BlockSpec((D, D), lambda i: (0, 0)),
            pl.BlockSpec((1, D), lambda i: (0, 0)),
        ],
        out_specs=[
            pl.BlockSpec((RBLK, D), lambda i: (i, 0)),
            pl.BlockSpec((8, D), lambda i: (0, 0)),
        ],
        out_shape=[
            jax.ShapeDtypeStruct((N, D), jnp.float32),
            jax.ShapeDtypeStruct((8, D), jnp.float32),
        ],
    )(X, W, b.reshape(1, D))
    H2 = pl.pallas_call(
        _k1b_body,
        grid=(nb,),
        in_specs=[
            pl.BlockSpec((RBLK, D), lambda i: (i, 0)),
            pl.BlockSpec((8, D), lambda i: (0, 0)),
            pl.BlockSpec((1, D), lambda i: (0, 0)),
            pl.BlockSpec((1, D), lambda i: (0, 0)),
        ],
        out_specs=pl.BlockSpec((2, RBLK, 128), lambda i: (0, i, 0)),
        out_shape=jax.ShapeDtypeStruct((2, N, 128), jnp.float32),
    )(H, S, gamma.reshape(1, D), beta.reshape(1, D))
    return H2


def _gs_pipeline(src, idx_v, out_idx_v, dst_sh, rows_a, rows_b, sem_a, sem_b):
    def _wait(rows, sem):
        pltpu.make_async_copy(src.at[idx_v.at[0]], rows, sem).wait()

    pltpu.async_copy(src.at[idx_v.at[0]], rows_a, sem_a)

    def _pair(i, _):
        ja = 2 * i
        pltpu.async_copy(src.at[idx_v.at[ja + 1]], rows_b, sem_b)
        _wait(rows_a, sem_a)
        pltpu.sync_copy(rows_a, dst_sh.at[out_idx_v.at[ja]], add=True)
        pltpu.async_copy(src.at[idx_v.at[(ja + 2) % CHUNKS]], rows_a, sem_a)
        _wait(rows_b, sem_b)
        pltpu.sync_copy(rows_b, dst_sh.at[out_idx_v.at[ja + 1]], add=True)
        return 0

    lax.fori_loop(0, CHUNKS // 2, _pair, 0)
    _wait(rows_a, sem_a)


def _sc_v2e_body(h2, vA, eA, acc_out, vidx_v, eidx_v, rows_a, rows_b,
                 zbuf_v, acc_sh, sem_a, sem_b):
    c = lax.axis_index("c")
    s = lax.axis_index("s")
    cN = c * N

    def _zero_row(j, _):
        for k in range(8):
            zbuf_v[j, k * 16:(k + 1) * 16] = jnp.zeros((16,), jnp.float32)
        return 0
    lax.fori_loop(0, MROWS, _zero_row, 0)

    def _group(g, _):
        pltpu.sync_copy(zbuf_v, acc_sh.at[pl.ds(s * MROWS, MROWS)])
        plsc.subcore_barrier()

        pltpu.sync_copy(vA.at[g, pl.ds(s * CHUNKS, CHUNKS)], vidx_v)
        pltpu.sync_copy(eA.at[g, pl.ds(s * CHUNKS, CHUNKS)], eidx_v)

        def _off_row(j, _):
            for k in range(8):
                sl = pl.ds(k * 16, 16)
                vidx_v[j, sl] = vidx_v[j, sl] + cN
            return 0
        lax.fori_loop(0, CHUNKS, _off_row, 0)

        def _chunk(j, _):
            pltpu.async_copy(h2.at[vidx_v.at[j]], rows_a, sem_a).wait()
            pltpu.sync_copy(rows_a, acc_sh.at[eidx_v.at[j]], add=True)
            return 0
        lax.fori_loop(0, CHUNKS, _chunk, 0)
        plsc.subcore_barrier()

        pltpu.sync_copy(acc_sh.at[pl.ds(s * MROWS, MROWS)],
                        acc_out.at[c, g, pl.ds(s * MROWS, MROWS)])
        return 0

    lax.fori_loop(0, 3, _group, 0)


def _make_sc_v2e():
    mesh = plsc.VectorSubcoreMesh(core_axis_name="c", subcore_axis_name="s")
    return pl.kernel(
        _sc_v2e_body,
        mesh=mesh,
        compiler_params=pltpu.CompilerParams(use_tc_tiling_on_sc=False),
        out_type=jax.ShapeDtypeStruct((2, 3, M_PAD, 128), jnp.float32),
        scratch_types=[
            pltpu.VMEM((CHUNKS, 128), jnp.int32),
            pltpu.VMEM((CHUNKS, 128), jnp.int32),
            pltpu.VMEM((128, 128), jnp.float32),
            pltpu.VMEM((128, 128), jnp.float32),
            pltpu.VMEM((MROWS, 128), jnp.float32),
            pltpu.VMEM_SHARED((M_PAD, 128), jnp.float32),
            pltpu.SemaphoreType.DMA,
            pltpu.SemaphoreType.DMA,
        ],
    )


HROWS = M_PAD * 16 // 128


def _sc_cnt_body(eA, cnt_out, eidx_v, hist_v, sem):
    c = lax.axis_index("c")
    s = lax.axis_index("s")
    del sem
    lanes = lax.iota(jnp.int32, 16)
    ones16 = jnp.ones((16,), jnp.float32)

    for g in range(3):
        def _zero(j, _):
            for k in range(8):
                hist_v[j, k * 16:(k + 1) * 16] = jnp.zeros((16,), jnp.float32)
            return 0
        lax.fori_loop(0, HROWS, _zero, 0)

        pltpu.sync_copy(
            eA.at[g, pl.ds((c * NTILES + s) * CCHUNKS, CCHUNKS)], eidx_v)

        def _chunk(j, _):
            for k in range(8):
                ev = eidx_v[j, k * 16:(k + 1) * 16]
                flat = ev * 16 + lanes
                plsc.addupdate_scatter(
                    hist_v, [lax.shift_right_logical(flat, 7), flat & 127],
                    ones16)
            return 0
        lax.fori_loop(0, CCHUNKS, _chunk, 0)

        pltpu.sync_copy(hist_v, cnt_out.at[c, s, g])


def _make_sc_cnt():
    mesh = plsc.VectorSubcoreMesh(core_axis_name="c", subcore_axis_name="s")
    return pl.kernel(
        _sc_cnt_body,
        mesh=mesh,
        compiler_params=pltpu.CompilerParams(needs_layout_passes=False),
        out_type=jax.ShapeDtypeStruct((2, NTILES, 3, HROWS, 128), jnp.float32),
        scratch_types=[
            pltpu.VMEM((CCHUNKS, 128), jnp.int32),
            pltpu.VMEM((HROWS, 128), jnp.float32),
            pltpu.SemaphoreType.DMA,
        ],
    )


def _kb_body(acc_ref, cnt_ref, w_ref, a_ref, z_ref):
    ch = jnp.sum(cnt_ref[...], axis=(0, 1, 2))
    ri = lax.broadcasted_iota(jnp.int32, (MBLK, MBLK // 8), 0)
    ci = lax.broadcasted_iota(jnp.int32, (MBLK, MBLK // 8), 1)
    U = (ri // 8 == ci).astype(jnp.float32)
    chr_ = jnp.dot(U, ch, preferred_element_type=jnp.float32)
    li = lax.broadcasted_iota(jnp.int32, (MBLK, 128), 1)
    ii = lax.broadcasted_iota(jnp.int32, (MBLK, 128), 0)
    sel = (li // 16) == (ii % 8)
    cnt = jnp.sum(jnp.where(sel, chr_, 0.0), axis=1, keepdims=True)
    cnt = jnp.maximum(cnt, 1.0)
    y0 = acc_ref[0, 0] / cnt
    y1 = acc_ref[1, 0] / cnt
    alpha = (jnp.dot(y0, w_ref[0], preferred_element_type=jnp.float32)
             + jnp.dot(y1, w_ref[1], preferred_element_type=jnp.float32))
    t = jnp.where(alpha >= 0, alpha, 0.2 * alpha)
    t = jnp.clip(t, 0.0, 5.0) * a_ref[pl.program_id(0), 0]
    z0 = y0 * t
    z1 = y1 * t
    z_ref[0, 0, 0] = z0[:, :64]
    z_ref[0, 1, 0] = z0[:, 64:]
    z_ref[1, 0, 0] = z1[:, :64]
    z_ref[1, 1, 0] = z1[:, 64:]


def _stage3(Acc, Cnt, w_e, avec):
    mb = M_PAD // MBLK
    return pl.pallas_call(
        _kb_body,
        grid=(3, mb),
        in_specs=[
            pl.BlockSpec((2, 1, MBLK, 128), lambda g, m: (0, g, m, 0)),
            pl.BlockSpec((2, NTILES, 1, MBLK * 16 // 128, 128),
                         lambda g, m: (0, 0, g, m, 0)),
            pl.BlockSpec((2, 128, 1), lambda g, m: (0, 0, 0)),
            pl.BlockSpec(memory_space=pltpu.SMEM),
        ],
        out_specs=pl.BlockSpec((2, 2, 1, MBLK, 64), lambda g, m: (0, 0, g, m, 0)),
        out_shape=jax.ShapeDtypeStruct((2, 2, 3, M_PAD, 64), jnp.float32),
    )(Acc, Cnt, w_e.reshape(2, 128, 1), avec)


def _sc_e2v_body(zflat, vC, eC, x_out, vidx_v, eidx_v, rows_a, rows_b,
                 zbuf_v, xacc_sh, sem_a, sem_b):
    c = lax.axis_index("c")
    s = lax.axis_index("s")
    XROWS = N_PAD // NTILES

    def _zero_row(j, _):
        for k in range(4):
            zbuf_v[j, k * 16:(k + 1) * 16] = jnp.zeros((16,), jnp.float32)
        return 0
    lax.fori_loop(0, XROWS, _zero_row, 0)

    for p in range(2):
        pltpu.sync_copy(zbuf_v, xacc_sh.at[pl.ds(s * XROWS, XROWS)])
        plsc.subcore_barrier()

        for g in range(3):
            plane = ((c * 2 + p) * 3 + g) * M_PAD
            pltpu.sync_copy(vC.at[g, pl.ds(s * CHUNKS, CHUNKS)], vidx_v)
            pltpu.sync_copy(eC.at[g, pl.ds(s * CHUNKS, CHUNKS)], eidx_v)

            def _off_row(j, _):
                for k in range(8):
                    sl = pl.ds(k * 16, 16)
                    eidx_v[j, sl] = eidx_v[j, sl] + plane
                return 0
            lax.fori_loop(0, CHUNKS, _off_row, 0)

            _gs_pipeline(zflat, eidx_v, vidx_v, xacc_sh, rows_a, rows_b,
                         sem_a, sem_b)
        plsc.subcore_barrier()

        pltpu.sync_copy(xacc_sh.at[pl.ds(s * XROWS, XROWS)],
                        x_out.at[c, p, pl.ds(s * XROWS, XROWS)])
        plsc.subcore_barrier()


def _make_sc_e2v():
    mesh = plsc.VectorSubcoreMesh(core_axis_name="c", subcore_axis_name="s")
    return pl.kernel(
        _sc_e2v_body,
        mesh=mesh,
        compiler_params=pltpu.CompilerParams(use_tc_tiling_on_sc=False),
        out_type=jax.ShapeDtypeStruct((2, 2, N_PAD, 64), jnp.float32),
        scratch_types=[
            pltpu.VMEM((CHUNKS, 128), jnp.int32),
            pltpu.VMEM((CHUNKS, 128), jnp.int32),
            pltpu.VMEM((128, 64), jnp.float32),
            pltpu.VMEM((128, 64), jnp.float32),
            pltpu.VMEM((N_PAD // NTILES, 64), jnp.float32),
            pltpu.VMEM_SHARED((N_PAD, 64), jnp.float32),
            pltpu.SemaphoreType.DMA,
            pltpu.SemaphoreType.DMA,
        ],
    )


def _kd_body(x_ref, o_ref):
    for c in range(2):
        for p in range(2):
            xq = x_ref[c, p]
            lo = (c * 2 + p) * 64
            o_ref[:, lo:lo + 64] = jnp.where(xq >= 0, xq, 0.01 * xq)


def _stage5(Xout):
    nb = N // RBLK
    return pl.pallas_call(
        _kd_body,
        grid=(nb,),
        in_specs=[pl.BlockSpec((2, 2, RBLK, 64), lambda i: (0, 0, i, 0))],
        out_specs=pl.BlockSpec((RBLK, D), lambda i: (i, 0)),
        out_shape=jax.ShapeDtypeStruct((N, D), jnp.float32),
    )(Xout)


def _pad_idx(v, e):
    ar = jnp.arange(E_PAD - E, dtype=jnp.int32)
    vA = jnp.concatenate([v, ar % 64])
    eA = jnp.concatenate([e, M + ar % (M_PAD - M)])
    vC = jnp.concatenate([v, N + ar % (N_PAD - N)])
    return vA, eA, vC


def kernel(X, v_hier, e_hier, v_cooc, e_cooc, v_cite, e_cite,
           W, b, gamma, beta, w_e, a1, a2, a3):
    groups = [(v_hier, e_hier), (v_cooc, e_cooc), (v_cite, e_cite)]
    vAs, eAs, vCs = [], [], []
    for v, e in groups:
        vA, eA, vC = _pad_idx(v, e)
        vAs.append(vA)
        eAs.append(eA)
        vCs.append(vC)
    vA3 = jnp.stack(vAs).reshape(3, E_PAD // 128, 128)
    eA3 = jnp.stack(eAs).reshape(3, E_PAD // 128, 128)
    vC3 = jnp.stack(vCs).reshape(3, E_PAD // 128, 128)

    H2 = _stage1(X, W, b, gamma, beta).reshape(2 * N, 128)
    Acc = _make_sc_v2e()(H2, vA3, eA3)
    Cnt = _make_sc_cnt()(eA3)
    avec = jnp.concatenate([a1.ravel(), a2.ravel(), a3.ravel()]).reshape(3, 1)
    Z = _stage3(Acc, Cnt, w_e, avec)
    Xout = _make_sc_e2v()(Z.reshape(12 * M_PAD, 64), vC3, eA3)
    Xo = _stage5(Xout)
    a = jnp.concatenate([a1.ravel(), a2.ravel(), a3.ravel()])
    return (Xo, a)


# speedup vs baseline: 8.8316x; 1.2435x over previous
import jax
import jax.numpy as jnp
from jax import lax
from jax.experimental import pallas as pl
from jax.experimental.pallas import tpu as pltpu
from jax.experimental.pallas import tpu_sc as plsc

N = 10000
D = 256
M = 5000
E = 160000

NTILES = 16
E_PAD = 163840
CHUNKS = E_PAD // NTILES // 128
CCHUNKS = CHUNKS // 2
M_PAD = 5120
N_PAD = 10240
NHALF = N_PAD // 2
XTRASH = 128
XTOT = NHALF + XTRASH
MROWS = M_PAD // NTILES
XZROWS = XTOT // NTILES
XCROWS = NHALF // NTILES
RBLK = 1000
MBLK = 640


def _k1a_body(x_ref, w_ref, b_ref, h_ref, s_ref):
    h = jnp.dot(x_ref[...], w_ref[...], preferred_element_type=jnp.float32)
    h = h + b_ref[...]
    h_ref[...] = h

    @pl.when(pl.program_id(0) == 0)
    def _():
        s_ref[...] = jnp.zeros_like(s_ref)

    s_ref[0:1, :] += jnp.sum(h, axis=0, keepdims=True)
    s_ref[1:2, :] += jnp.sum(h * h, axis=0, keepdims=True)


def _k1b_body(h_ref, s_ref, g_ref, be_ref, o_ref):
    mu = s_ref[0:1, :] / N
    var = s_ref[1:2, :] / N - mu * mu
    hn = (h_ref[...] - mu) * (lax.rsqrt(var + 1e-5) * g_ref[...]) + be_ref[...]
    o_ref[0] = hn[:, :128]
    o_ref[1] = hn[:, 128:]


def _stage1(X, W, b, gamma, beta):
    nb = N // RBLK
    H, S = pl.pallas_call(
        _k1a_body,
        grid=(nb,),
        in_specs=[
            pl.BlockSpec((RBLK, D), lambda i: (i, 0)),
            pl.BlockSpec((D, D), lambda i: (0, 0)),
            pl.BlockSpec((1, D), lambda i: (0, 0)),
        ],
        out_specs=[
            pl.BlockSpec((RBLK, D), lambda i: (i, 0)),
            pl.BlockSpec((8, D), lambda i: (0, 0)),
        ],
        out_shape=[
            jax.ShapeDtypeStruct((N, D), jnp.float32),
            jax.ShapeDtypeStruct((8, D), jnp.float32),
        ],
    )(X, W, b.reshape(1, D))
    H2 = pl.pallas_call(
        _k1b_body,
        grid=(nb,),
        in_specs=[
            pl.BlockSpec((RBLK, D), lambda i: (i, 0)),
            pl.BlockSpec((8, D), lambda i: (0, 0)),
            pl.BlockSpec((1, D), lambda i: (0, 0)),
            pl.BlockSpec((1, D), lambda i: (0, 0)),
        ],
        out_specs=pl.BlockSpec((2, RBLK, 128), lambda i: (0, i, 0)),
        out_shape=jax.ShapeDtypeStruct((2, N, 128), jnp.float32),
    )(H, S, gamma.reshape(1, D), beta.reshape(1, D))
    return H2


def _gs_pipeline(src, idx_v, out_idx_v, dst_sh, rows_a, rows_b, sem_a, sem_b):
    def _wait(rows, sem):
        pltpu.make_async_copy(src.at[idx_v.at[0]], rows, sem).wait()

    pltpu.async_copy(src.at[idx_v.at[0]], rows_a, sem_a)

    def _pair(i, _):
        ja = 2 * i
        pltpu.async_copy(src.at[idx_v.at[ja + 1]], rows_b, sem_b)
        _wait(rows_a, sem_a)
        pltpu.sync_copy(rows_a, dst_sh.at[out_idx_v.at[ja]], add=True)
        pltpu.async_copy(src.at[idx_v.at[(ja + 2) % CHUNKS]], rows_a, sem_a)
        _wait(rows_b, sem_b)
        pltpu.sync_copy(rows_b, dst_sh.at[out_idx_v.at[ja + 1]], add=True)
        return 0

    lax.fori_loop(0, CHUNKS // 2, _pair, 0)
    _wait(rows_a, sem_a)


def _sc_v2e_body(h2, vA, eA, acc_out, vidx_v, eidx_v, rows_a, rows_b,
                 zbuf_v, acc_sh, sem_a, sem_b):
    c = lax.axis_index("c")
    s = lax.axis_index("s")
    cN = c * N

    def _zero_row(j, _):
        for k in range(8):
            zbuf_v[j, k * 16:(k + 1) * 16] = jnp.zeros((16,), jnp.float32)
        return 0
    lax.fori_loop(0, MROWS // 4, _zero_row, 0)

    def _group(g, _):
        for q in range(4):
            pltpu.sync_copy(
                zbuf_v, acc_sh.at[pl.ds(s * MROWS + q * (MROWS // 4),
                                        MROWS // 4)])
        plsc.subcore_barrier()

        pltpu.sync_copy(vA.at[g, pl.ds(s * CHUNKS, CHUNKS)], vidx_v)
        pltpu.sync_copy(eA.at[g, pl.ds(s * CHUNKS, CHUNKS)], eidx_v)

        def _off_row(j, _):
            for k in range(8):
                sl = pl.ds(k * 16, 16)
                vidx_v[j, sl] = vidx_v[j, sl] + cN
            return 0
        lax.fori_loop(0, CHUNKS, _off_row, 0)

        _gs_pipeline(h2, vidx_v, eidx_v, acc_sh, rows_a, rows_b, sem_a, sem_b)
        plsc.subcore_barrier()

        pltpu.sync_copy(acc_sh.at[pl.ds(s * MROWS, MROWS)],
                        acc_out.at[c, g, pl.ds(s * MROWS, MROWS)])
        return 0

    lax.fori_loop(0, 3, _group, 0)


def _make_sc_v2e():
    mesh = plsc.VectorSubcoreMesh(core_axis_name="c", subcore_axis_name="s")
    return pl.kernel(
        _sc_v2e_body,
        mesh=mesh,
        compiler_params=pltpu.CompilerParams(use_tc_tiling_on_sc=False),
        out_type=jax.ShapeDtypeStruct((2, 3, M_PAD, 128), jnp.float32),
        scratch_types=[
            pltpu.VMEM((CHUNKS, 128), jnp.int32),
            pltpu.VMEM((CHUNKS, 128), jnp.int32),
            pltpu.VMEM((128, 128), jnp.float32),
            pltpu.VMEM((128, 128), jnp.float32),
            pltpu.VMEM((MROWS // 4, 128), jnp.float32),
            pltpu.VMEM_SHARED((M_PAD, 128), jnp.float32),
            pltpu.SemaphoreType.DMA,
            pltpu.SemaphoreType.DMA,
        ],
    )


HROWS = M_PAD * 16 // 128


def _sc_cnt_body(eA, cnt_out, eidx_v, hist_v, sem):
    c = lax.axis_index("c")
    s = lax.axis_index("s")
    del sem
    lanes = lax.iota(jnp.int32, 16)
    ones16 = jnp.ones((16,), jnp.float32)

    for g in range(3):
        def _zero(j, _):
            for k in range(8):
                hist_v[j, k * 16:(k + 1) * 16] = jnp.zeros((16,), jnp.float32)
            return 0
        lax.fori_loop(0, HROWS, _zero, 0)

        pltpu.sync_copy(
            eA.at[g, pl.ds((c * NTILES + s) * CCHUNKS, CCHUNKS)], eidx_v)

        def _chunk(j, _):
            for k in range(8):
                ev = eidx_v[j, k * 16:(k + 1) * 16]
                flat = ev * 16 + lanes
                plsc.addupdate_scatter(
                    hist_v, [lax.shift_right_logical(flat, 7), flat & 127],
                    ones16)
            return 0
        lax.fori_loop(0, CCHUNKS, _chunk, 0)

        pltpu.sync_copy(hist_v, cnt_out.at[c, s, g])


def _make_sc_cnt():
    mesh = plsc.VectorSubcoreMesh(core_axis_name="c", subcore_axis_name="s")
    return pl.kernel(
        _sc_cnt_body,
        mesh=mesh,
        compiler_params=pltpu.CompilerParams(needs_layout_passes=False),
        out_type=jax.ShapeDtypeStruct((2, NTILES, 3, HROWS, 128), jnp.float32),
        scratch_types=[
            pltpu.VMEM((CCHUNKS, 128), jnp.int32),
            pltpu.VMEM((HROWS, 128), jnp.float32),
            pltpu.SemaphoreType.DMA,
        ],
    )


def _kb_body(acc_ref, cnt_ref, w_ref, a_ref, z_ref):
    ch = jnp.sum(cnt_ref[...], axis=(0, 1, 2))
    ri = lax.broadcasted_iota(jnp.int32, (MBLK, MBLK // 8), 0)
    ci = lax.broadcasted_iota(jnp.int32, (MBLK, MBLK // 8), 1)
    U = (ri // 8 == ci).astype(jnp.float32)
    chr_ = jnp.dot(U, ch, preferred_element_type=jnp.float32)
    li = lax.broadcasted_iota(jnp.int32, (MBLK, 128), 1)
    ii = lax.broadcasted_iota(jnp.int32, (MBLK, 128), 0)
    sel = (li // 16) == (ii % 8)
    cnt = jnp.sum(jnp.where(sel, chr_, 0.0), axis=1, keepdims=True)
    cnt = jnp.maximum(cnt, 1.0)
    y0 = acc_ref[0, 0] / cnt
    y1 = acc_ref[1, 0] / cnt
    alpha = (jnp.dot(y0, w_ref[0], preferred_element_type=jnp.float32)
             + jnp.dot(y1, w_ref[1], preferred_element_type=jnp.float32))
    t = jnp.where(alpha >= 0, alpha, 0.2 * alpha)
    t = jnp.clip(t, 0.0, 5.0) * a_ref[pl.program_id(0), 0]
    z0 = y0 * t
    z1 = y1 * t
    z_ref[0, 0, 0] = z0[:, :64]
    z_ref[0, 1, 0] = z0[:, 64:]
    z_ref[1, 0, 0] = z1[:, :64]
    z_ref[1, 1, 0] = z1[:, 64:]


def _stage3(Acc, Cnt, w_e, avec):
    mb = M_PAD // MBLK
    return pl.pallas_call(
        _kb_body,
        grid=(3, mb),
        in_specs=[
            pl.BlockSpec((2, 1, MBLK, 128), lambda g, m: (0, g, m, 0)),
            pl.BlockSpec((2, NTILES, 1, MBLK * 16 // 128, 128),
                         lambda g, m: (0, 0, g, m, 0)),
            pl.BlockSpec((2, 128, 1), lambda g, m: (0, 0, 0)),
            pl.BlockSpec(memory_space=pltpu.SMEM),
        ],
        out_specs=pl.BlockSpec((2, 2, 1, MBLK, 64), lambda g, m: (0, 0, g, m, 0)),
        out_shape=jax.ShapeDtypeStruct((2, 2, 3, M_PAD, 64), jnp.float32),
    )(Acc, Cnt, w_e.reshape(2, 128, 1), avec)


def _sc_e2v_body(zflat, vC, eC, x_out, vidx_v, eidx_v, rows_a, rows_b,
                 zbuf_v, xacc_sh, sem_a, sem_b):
    c = lax.axis_index("c")
    s = lax.axis_index("s")
    XROWS = N_PAD // NTILES

    def _zero_row(j, _):
        for k in range(4):
            zbuf_v[j, k * 16:(k + 1) * 16] = jnp.zeros((16,), jnp.float32)
        return 0
    lax.fori_loop(0, XROWS, _zero_row, 0)

    for p in range(2):
        pltpu.sync_copy(zbuf_v, xacc_sh.at[pl.ds(s * XROWS, XROWS)])
        plsc.subcore_barrier()

        for g in range(3):
            plane = ((c * 2 + p) * 3 + g) * M_PAD
            pltpu.sync_copy(vC.at[g, pl.ds(s * CHUNKS, CHUNKS)], vidx_v)
            pltpu.sync_copy(eC.at[g, pl.ds(s * CHUNKS, CHUNKS)], eidx_v)

            def _off_row(j, _):
                for k in range(8):
                    sl = pl.ds(k * 16, 16)
                    eidx_v[j, sl] = eidx_v[j, sl] + plane
                return 0
            lax.fori_loop(0, CHUNKS, _off_row, 0)

            _gs_pipeline(zflat, eidx_v, vidx_v, xacc_sh, rows_a, rows_b,
                         sem_a, sem_b)
        plsc.subcore_barrier()

        pltpu.sync_copy(xacc_sh.at[pl.ds(s * XROWS, XROWS)],
                        x_out.at[c, p, pl.ds(s * XROWS, XROWS)])
        plsc.subcore_barrier()


def _make_sc_e2v():
    mesh = plsc.VectorSubcoreMesh(core_axis_name="c", subcore_axis_name="s")
    return pl.kernel(
        _sc_e2v_body,
        mesh=mesh,
        compiler_params=pltpu.CompilerParams(use_tc_tiling_on_sc=False),
        out_type=jax.ShapeDtypeStruct((2, 2, N_PAD, 64), jnp.float32),
        scratch_types=[
            pltpu.VMEM((CHUNKS, 128), jnp.int32),
            pltpu.VMEM((CHUNKS, 128), jnp.int32),
            pltpu.VMEM((128, 64), jnp.float32),
            pltpu.VMEM((128, 64), jnp.float32),
            pltpu.VMEM((N_PAD // NTILES, 64), jnp.float32),
            pltpu.VMEM_SHARED((N_PAD, 64), jnp.float32),
            pltpu.SemaphoreType.DMA,
            pltpu.SemaphoreType.DMA,
        ],
    )


def _kd_body(x_ref, o_ref):
    for c in range(2):
        for p in range(2):
            xq = x_ref[c, p]
            lo = (c * 2 + p) * 64
            o_ref[:, lo:lo + 64] = jnp.where(xq >= 0, xq, 0.01 * xq)


def _stage5(Xout):
    nb = N // RBLK
    return pl.pallas_call(
        _kd_body,
        grid=(nb,),
        in_specs=[pl.BlockSpec((2, 2, RBLK, 64), lambda i: (0, 0, i, 0))],
        out_specs=pl.BlockSpec((RBLK, D), lambda i: (i, 0)),
        out_shape=jax.ShapeDtypeStruct((N, D), jnp.float32),
    )(Xout)


def _pad_idx(v, e):
    ar = jnp.arange(E_PAD - E, dtype=jnp.int32)
    vA = jnp.concatenate([v, ar % 64])
    eA = jnp.concatenate([e, M + ar % (M_PAD - M)])
    vC = jnp.concatenate([v, N + ar % (N_PAD - N)])
    return vA, eA, vC


def kernel(X, v_hier, e_hier, v_cooc, e_cooc, v_cite, e_cite,
           W, b, gamma, beta, w_e, a1, a2, a3):
    groups = [(v_hier, e_hier), (v_cooc, e_cooc), (v_cite, e_cite)]
    vAs, eAs, vCs = [], [], []
    for v, e in groups:
        vA, eA, vC = _pad_idx(v, e)
        vAs.append(vA)
        eAs.append(eA)
        vCs.append(vC)
    vA3 = jnp.stack(vAs).reshape(3, E_PAD // 128, 128)
    eA3 = jnp.stack(eAs).reshape(3, E_PAD // 128, 128)
    vC3 = jnp.stack(vCs).reshape(3, E_PAD // 128, 128)

    H2 = _stage1(X, W, b, gamma, beta).reshape(2 * N, 128)
    Acc = _make_sc_v2e()(H2, vA3, eA3)
    Cnt = _make_sc_cnt()(eA3)
    avec = jnp.concatenate([a1.ravel(), a2.ravel(), a3.ravel()]).reshape(3, 1)
    Z = _stage3(Acc, Cnt, w_e, avec)
    Xout = _make_sc_e2v()(Z.reshape(12 * M_PAD, 64), vC3, eA3)
    Xo = _stage5(Xout)
    a = jnp.concatenate([a1.ravel(), a2.ravel(), a3.ravel()])
    return (Xo, a)


# speedup vs baseline: 8.9354x; 1.0118x over previous
import jax
import jax.numpy as jnp
from jax import lax
from jax.experimental import pallas as pl
from jax.experimental.pallas import tpu as pltpu
from jax.experimental.pallas import tpu_sc as plsc

N = 10000
D = 256
M = 5000
E = 160000

NTILES = 16
E_PAD = 163840
CHUNKS = E_PAD // NTILES // 128
CCHUNKS = CHUNKS // 2
M_PAD = 5120
N_PAD = 10240
NHALF = N_PAD // 2
XTRASH = 128
XTOT = NHALF + XTRASH
MROWS = M_PAD // NTILES
XZROWS = XTOT // NTILES
XCROWS = NHALF // NTILES
RBLK = 1000
MBLK = 640


NB = N // RBLK


def _k1_body(x_ref, w_ref, b_ref, g_ref, be_ref, o_ref, h_scr, s_scr):
    i = pl.program_id(0)

    @pl.when(i < NB)
    def _():
        h = jnp.dot(x_ref[...], w_ref[...],
                    preferred_element_type=jnp.float32)
        h = h + b_ref[...]
        h_scr[pl.ds(i * RBLK, RBLK), :] = h

        @pl.when(i == 0)
        def _():
            s_scr[...] = jnp.zeros_like(s_scr)

        s_scr[0:1, :] += jnp.sum(h, axis=0, keepdims=True)
        s_scr[1:2, :] += jnp.sum(h * h, axis=0, keepdims=True)

    @pl.when(i == NB)
    def _():
        mu = s_scr[0:1, :] / N
        var = s_scr[1:2, :] / N - mu * mu
        hn = ((h_scr[...] - mu) * (lax.rsqrt(var + 1e-5) * g_ref[...])
              + be_ref[...])
        o_ref[0] = hn[:, :128]
        o_ref[1] = hn[:, 128:]


def _stage1(X, W, b, gamma, beta):
    return pl.pallas_call(
        _k1_body,
        grid=(NB + 1,),
        in_specs=[
            pl.BlockSpec((RBLK, D), lambda i: (jnp.minimum(i, NB - 1), 0)),
            pl.BlockSpec((D, D), lambda i: (0, 0)),
            pl.BlockSpec((1, D), lambda i: (0, 0)),
            pl.BlockSpec((1, D), lambda i: (0, 0)),
            pl.BlockSpec((1, D), lambda i: (0, 0)),
        ],
        out_specs=pl.BlockSpec((2, N, 128), lambda i: (0, 0, 0)),
        out_shape=jax.ShapeDtypeStruct((2, N, 128), jnp.float32),
        scratch_shapes=[
            pltpu.VMEM((N, D), jnp.float32),
            pltpu.VMEM((8, D), jnp.float32),
        ],
    )(X, W, b.reshape(1, D), gamma.reshape(1, D), beta.reshape(1, D))


def _gs_pipeline(src, idx_v, out_idx_v, dst_sh, rows_a, rows_b, sem_a, sem_b):
    def _wait(rows, sem):
        pltpu.make_async_copy(src.at[idx_v.at[0]], rows, sem).wait()

    pltpu.async_copy(src.at[idx_v.at[0]], rows_a, sem_a)

    def _pair(i, _):
        ja = 2 * i
        pltpu.async_copy(src.at[idx_v.at[ja + 1]], rows_b, sem_b)
        _wait(rows_a, sem_a)
        pltpu.sync_copy(rows_a, dst_sh.at[out_idx_v.at[ja]], add=True)
        pltpu.async_copy(src.at[idx_v.at[(ja + 2) % CHUNKS]], rows_a, sem_a)
        _wait(rows_b, sem_b)
        pltpu.sync_copy(rows_b, dst_sh.at[out_idx_v.at[ja + 1]], add=True)
        return 0

    lax.fori_loop(0, CHUNKS // 2, _pair, 0)
    _wait(rows_a, sem_a)


def _sc_v2e_body(h2, vA, eA, acc_out, vidx_v, eidx_v, rows_a, rows_b,
                 zbuf_v, acc_sh, sem_a, sem_b):
    c = lax.axis_index("c")
    s = lax.axis_index("s")
    cN = c * N

    def _zero_row(j, _):
        for k in range(8):
            zbuf_v[j, k * 16:(k + 1) * 16] = jnp.zeros((16,), jnp.float32)
        return 0
    lax.fori_loop(0, MROWS // 4, _zero_row, 0)

    def _group(g, _):
        for q in range(4):
            pltpu.sync_copy(
                zbuf_v, acc_sh.at[pl.ds(s * MROWS + q * (MROWS // 4),
                                        MROWS // 4)])
        plsc.subcore_barrier()

        pltpu.sync_copy(vA.at[g, pl.ds(s * CHUNKS, CHUNKS)], vidx_v)
        pltpu.sync_copy(eA.at[g, pl.ds(s * CHUNKS, CHUNKS)], eidx_v)

        def _off_row(j, _):
            for k in range(8):
                sl = pl.ds(k * 16, 16)
                vidx_v[j, sl] = vidx_v[j, sl] + cN
            return 0
        lax.fori_loop(0, CHUNKS, _off_row, 0)

        _gs_pipeline(h2, vidx_v, eidx_v, acc_sh, rows_a, rows_b, sem_a, sem_b)
        plsc.subcore_barrier()

        pltpu.sync_copy(acc_sh.at[pl.ds(s * MROWS, MROWS)],
                        acc_out.at[c, g, pl.ds(s * MROWS, MROWS)])
        return 0

    lax.fori_loop(0, 3, _group, 0)


def _make_sc_v2e():
    mesh = plsc.VectorSubcoreMesh(core_axis_name="c", subcore_axis_name="s")
    return pl.kernel(
        _sc_v2e_body,
        mesh=mesh,
        compiler_params=pltpu.CompilerParams(use_tc_tiling_on_sc=False),
        out_type=jax.ShapeDtypeStruct((2, 3, M_PAD, 128), jnp.float32),
        scratch_types=[
            pltpu.VMEM((CHUNKS, 128), jnp.int32),
            pltpu.VMEM((CHUNKS, 128), jnp.int32),
            pltpu.VMEM((128, 128), jnp.float32),
            pltpu.VMEM((128, 128), jnp.float32),
            pltpu.VMEM((MROWS // 4, 128), jnp.float32),
            pltpu.VMEM_SHARED((M_PAD, 128), jnp.float32),
            pltpu.SemaphoreType.DMA,
            pltpu.SemaphoreType.DMA,
        ],
    )


HROWS = M_PAD * 16 // 128


def _sc_cnt_body(eA, cnt_out, eidx_v, hist_v, sem):
    c = lax.axis_index("c")
    s = lax.axis_index("s")
    del sem
    lanes = lax.iota(jnp.int32, 16)
    ones16 = jnp.ones((16,), jnp.float32)

    for g in range(3):
        def _zero(j, _):
            for k in range(8):
                hist_v[j, k * 16:(k + 1) * 16] = jnp.zeros((16,), jnp.float32)
            return 0
        lax.fori_loop(0, HROWS, _zero, 0)

        pltpu.sync_copy(
            eA.at[g, pl.ds((c * NTILES + s) * CCHUNKS, CCHUNKS)], eidx_v)

        def _chunk(j, _):
            for k in range(8):
                ev = eidx_v[j, k * 16:(k + 1) * 16]
                flat = ev * 16 + lanes
                plsc.addupdate_scatter(
                    hist_v, [lax.shift_right_logical(flat, 7), flat & 127],
                    ones16)
            return 0
        lax.fori_loop(0, CCHUNKS, _chunk, 0)

        pltpu.sync_copy(hist_v, cnt_out.at[c, s, g])


def _make_sc_cnt():
    mesh = plsc.VectorSubcoreMesh(core_axis_name="c", subcore_axis_name="s")
    return pl.kernel(
        _sc_cnt_body,
        mesh=mesh,
        compiler_params=pltpu.CompilerParams(needs_layout_passes=False),
        out_type=jax.ShapeDtypeStruct((2, NTILES, 3, HROWS, 128), jnp.float32),
        scratch_types=[
            pltpu.VMEM((CCHUNKS, 128), jnp.int32),
            pltpu.VMEM((HROWS, 128), jnp.float32),
            pltpu.SemaphoreType.DMA,
        ],
    )


def _kb_body(acc_ref, cnt_ref, w_ref, a_ref, z_ref):
    ch = jnp.sum(cnt_ref[...], axis=(0, 1, 2))
    ri = lax.broadcasted_iota(jnp.int32, (MBLK, MBLK // 8), 0)
    ci = lax.broadcasted_iota(jnp.int32, (MBLK, MBLK // 8), 1)
    U = (ri // 8 == ci).astype(jnp.float32)
    chr_ = jnp.dot(U, ch, preferred_element_type=jnp.float32)
    li = lax.broadcasted_iota(jnp.int32, (MBLK, 128), 1)
    ii = lax.broadcasted_iota(jnp.int32, (MBLK, 128), 0)
    sel = (li // 16) == (ii % 8)
    cnt = jnp.sum(jnp.where(sel, chr_, 0.0), axis=1, keepdims=True)
    cnt = jnp.maximum(cnt, 1.0)
    y0 = acc_ref[0, 0] / cnt
    y1 = acc_ref[1, 0] / cnt
    alpha = (jnp.dot(y0, w_ref[0], preferred_element_type=jnp.float32)
             + jnp.dot(y1, w_ref[1], preferred_element_type=jnp.float32))
    t = jnp.where(alpha >= 0, alpha, 0.2 * alpha)
    t = jnp.clip(t, 0.0, 5.0) * a_ref[pl.program_id(0), 0]
    z0 = y0 * t
    z1 = y1 * t
    z_ref[0, 0, 0] = z0[:, :64]
    z_ref[0, 1, 0] = z0[:, 64:]
    z_ref[1, 0, 0] = z1[:, :64]
    z_ref[1, 1, 0] = z1[:, 64:]


def _stage3(Acc, Cnt, w_e, avec):
    mb = M_PAD // MBLK
    return pl.pallas_call(
        _kb_body,
        grid=(3, mb),
        in_specs=[
            pl.BlockSpec((2, 1, MBLK, 128), lambda g, m: (0, g, m, 0)),
            pl.BlockSpec((2, NTILES, 1, MBLK * 16 // 128, 128),
                         lambda g, m: (0, 0, g, m, 0)),
            pl.BlockSpec((2, 128, 1), lambda g, m: (0, 0, 0)),
            pl.BlockSpec(memory_space=pltpu.SMEM),
        ],
        out_specs=pl.BlockSpec((2, 2, 1, MBLK, 64), lambda g, m: (0, 0, g, m, 0)),
        out_shape=jax.ShapeDtypeStruct((2, 2, 3, M_PAD, 64), jnp.float32),
    )(Acc, Cnt, w_e.reshape(2, 128, 1), avec)


def _sc_e2v_body(zflat, vC, eC, x_out, vidx_v, eidx_v, rows_a, rows_b,
                 zbuf_v, xacc_sh, sem_a, sem_b):
    c = lax.axis_index("c")
    s = lax.axis_index("s")
    XROWS = N_PAD // NTILES

    def _zero_row(j, _):
        for k in range(4):
            zbuf_v[j, k * 16:(k + 1) * 16] = jnp.zeros((16,), jnp.float32)
        return 0
    lax.fori_loop(0, XROWS, _zero_row, 0)

    for p in range(2):
        pltpu.sync_copy(zbuf_v, xacc_sh.at[pl.ds(s * XROWS, XROWS)])
        plsc.subcore_barrier()

        for g in range(3):
            plane = ((c * 2 + p) * 3 + g) * M_PAD
            pltpu.sync_copy(vC.at[g, pl.ds(s * CHUNKS, CHUNKS)], vidx_v)
            pltpu.sync_copy(eC.at[g, pl.ds(s * CHUNKS, CHUNKS)], eidx_v)

            def _off_row(j, _):
                for k in range(8):
                    sl = pl.ds(k * 16, 16)
                    eidx_v[j, sl] = eidx_v[j, sl] + plane
                return 0
            lax.fori_loop(0, CHUNKS, _off_row, 0)

            _gs_pipeline(zflat, eidx_v, vidx_v, xacc_sh, rows_a, rows_b,
                         sem_a, sem_b)
        plsc.subcore_barrier()

        pltpu.sync_copy(xacc_sh.at[pl.ds(s * XROWS, XROWS)],
                        x_out.at[c, p, pl.ds(s * XROWS, XROWS)])
        plsc.subcore_barrier()


def _make_sc_e2v():
    mesh = plsc.VectorSubcoreMesh(core_axis_name="c", subcore_axis_name="s")
    return pl.kernel(
        _sc_e2v_body,
        mesh=mesh,
        compiler_params=pltpu.CompilerParams(use_tc_tiling_on_sc=False),
        out_type=jax.ShapeDtypeStruct((2, 2, N_PAD, 64), jnp.float32),
        scratch_types=[
            pltpu.VMEM((CHUNKS, 128), jnp.int32),
            pltpu.VMEM((CHUNKS, 128), jnp.int32),
            pltpu.VMEM((128, 64), jnp.float32),
            pltpu.VMEM((128, 64), jnp.float32),
            pltpu.VMEM((N_PAD // NTILES, 64), jnp.float32),
            pltpu.VMEM_SHARED((N_PAD, 64), jnp.float32),
            pltpu.SemaphoreType.DMA,
            pltpu.SemaphoreType.DMA,
        ],
    )


def _kd_body(x_ref, o_ref):
    for c in range(2):
        for p in range(2):
            xq = x_ref[c, p]
            lo = (c * 2 + p) * 64
            o_ref[:, lo:lo + 64] = jnp.where(xq >= 0, xq, 0.01 * xq)


def _stage5(Xout):
    nb = N // RBLK
    return pl.pallas_call(
        _kd_body,
        grid=(nb,),
        in_specs=[pl.BlockSpec((2, 2, RBLK, 64), lambda i: (0, 0, i, 0))],
        out_specs=pl.BlockSpec((RBLK, D), lambda i: (i, 0)),
        out_shape=jax.ShapeDtypeStruct((N, D), jnp.float32),
    )(Xout)


def _pad_idx(v, e):
    ar = jnp.arange(E_PAD - E, dtype=jnp.int32)
    vA = jnp.concatenate([v, ar % 64])
    eA = jnp.concatenate([e, M + ar % (M_PAD - M)])
    vC = jnp.concatenate([v, N + ar % (N_PAD - N)])
    return vA, eA, vC


def kernel(X, v_hier, e_hier, v_cooc, e_cooc, v_cite, e_cite,
           W, b, gamma, beta, w_e, a1, a2, a3):
    groups = [(v_hier, e_hier), (v_cooc, e_cooc), (v_cite, e_cite)]
    vAs, eAs, vCs = [], [], []
    for v, e in groups:
        vA, eA, vC = _pad_idx(v, e)
        vAs.append(vA)
        eAs.append(eA)
        vCs.append(vC)
    vA3 = jnp.stack(vAs).reshape(3, E_PAD // 128, 128)
    eA3 = jnp.stack(eAs).reshape(3, E_PAD // 128, 128)
    vC3 = jnp.stack(vCs).reshape(3, E_PAD // 128, 128)

    Cnt = _make_sc_cnt()(eA3)
    H2 = _stage1(X, W, b, gamma, beta).reshape(2 * N, 128)
    Acc = _make_sc_v2e()(H2, vA3, eA3)
    avec = jnp.concatenate([a1.ravel(), a2.ravel(), a3.ravel()]).reshape(3, 1)
    Z = _stage3(Acc, Cnt, w_e, avec)
    Xout = _make_sc_e2v()(Z.reshape(12 * M_PAD, 64), vC3, eA3)
    Xo = _stage5(Xout)
    a = jnp.concatenate([a1.ravel(), a2.ravel(), a3.ravel()])
    return (Xo, a)


# speedup vs baseline: 10.0417x; 1.1238x over previous
import jax
import jax.numpy as jnp
from jax import lax
from jax.experimental import pallas as pl
from jax.experimental.pallas import tpu as pltpu
from jax.experimental.pallas import tpu_sc as plsc

N = 10000
D = 256
M = 5000
E = 160000

NTILES = 16
E_PAD = 163840
CHUNKS = E_PAD // NTILES // 128
CCHUNKS = CHUNKS // 2
M_PAD = 5120
N_PAD = 10240
NHALF = N_PAD // 2
XTRASH = 128
XTOT = NHALF + XTRASH
MROWS = M_PAD // NTILES
XZROWS = XTOT // NTILES
XCROWS = NHALF // NTILES
RBLK = 1000
MBLK = 640


NB = N // RBLK


def _k1_body(x_ref, w_ref, b_ref, g_ref, be_ref, o_ref, h_scr, s_scr):
    i = pl.program_id(0)

    @pl.when(i < NB)
    def _():
        h = jnp.dot(x_ref[...], w_ref[...],
                    preferred_element_type=jnp.float32)
        h = h + b_ref[...]
        h_scr[pl.ds(i * RBLK, RBLK), :] = h

        @pl.when(i == 0)
        def _():
            s_scr[...] = jnp.zeros_like(s_scr)

        s_scr[0:1, :] += jnp.sum(h, axis=0, keepdims=True)
        s_scr[1:2, :] += jnp.sum(h * h, axis=0, keepdims=True)

    @pl.when(i == NB)
    def _():
        mu = s_scr[0:1, :] / N
        var = s_scr[1:2, :] / N - mu * mu
        hn = ((h_scr[...] - mu) * (lax.rsqrt(var + 1e-5) * g_ref[...])
              + be_ref[...])
        o_ref[0] = hn[:, :128]
        o_ref[1] = hn[:, 128:]


def _stage1(X, W, b, gamma, beta):
    return pl.pallas_call(
        _k1_body,
        grid=(NB + 1,),
        in_specs=[
            pl.BlockSpec((RBLK, D), lambda i: (jnp.minimum(i, NB - 1), 0)),
            pl.BlockSpec((D, D), lambda i: (0, 0)),
            pl.BlockSpec((1, D), lambda i: (0, 0)),
            pl.BlockSpec((1, D), lambda i: (0, 0)),
            pl.BlockSpec((1, D), lambda i: (0, 0)),
        ],
        out_specs=pl.BlockSpec((2, N, 128), lambda i: (0, 0, 0)),
        out_shape=jax.ShapeDtypeStruct((2, N, 128), jnp.float32),
        scratch_shapes=[
            pltpu.VMEM((N, D), jnp.float32),
            pltpu.VMEM((8, D), jnp.float32),
        ],
    )(X, W, b.reshape(1, D), gamma.reshape(1, D), beta.reshape(1, D))


def _gs_pipeline(src, idx_v, out_idx_v, dst_sh, rows_a, rows_b, sem_a, sem_b):
    def _wait(rows, sem):
        pltpu.make_async_copy(src.at[idx_v.at[0]], rows, sem).wait()

    pltpu.async_copy(src.at[idx_v.at[0]], rows_a, sem_a)

    def _pair(i, _):
        ja = 2 * i
        pltpu.async_copy(src.at[idx_v.at[ja + 1]], rows_b, sem_b)
        _wait(rows_a, sem_a)
        pltpu.sync_copy(rows_a, dst_sh.at[out_idx_v.at[ja]], add=True)
        pltpu.async_copy(src.at[idx_v.at[(ja + 2) % CHUNKS]], rows_a, sem_a)
        _wait(rows_b, sem_b)
        pltpu.sync_copy(rows_b, dst_sh.at[out_idx_v.at[ja + 1]], add=True)
        return 0

    lax.fori_loop(0, CHUNKS // 2, _pair, 0)
    _wait(rows_a, sem_a)


def _gs_pipeline4(src, idx_v, out_idx_v, dst_sh, bufs, sems):
    def _wait(rows, sem):
        pltpu.make_async_copy(src.at[idx_v.at[0]], rows, sem).wait()

    for b in range(4):
        pltpu.async_copy(src.at[idx_v.at[b]], bufs[b], sems[b])

    def _quad(i, _):
        j = 4 * i
        for b in range(4):
            _wait(bufs[b], sems[b])
            pltpu.sync_copy(bufs[b], dst_sh.at[out_idx_v.at[j + b]], add=True)
            pltpu.async_copy(src.at[idx_v.at[(j + b + 4) % CHUNKS]],
                             bufs[b], sems[b])
        return 0

    lax.fori_loop(0, CHUNKS // 4, _quad, 0)
    for b in range(4):
        _wait(bufs[b], sems[b])


def _sc_v2e_body(h2, vA, eA, acc_out, vidx_v, eidx_v, rows_a, rows_b,
                 zbuf_v, acc_sh, sem_a, sem_b):
    c = lax.axis_index("c")
    s = lax.axis_index("s")
    cN = c * N

    def _zero_row(j, _):
        for k in range(8):
            zbuf_v[j, k * 16:(k + 1) * 16] = jnp.zeros((16,), jnp.float32)
        return 0
    lax.fori_loop(0, MROWS // 4, _zero_row, 0)

    def _group(g, _):
        for q in range(4):
            pltpu.sync_copy(
                zbuf_v, acc_sh.at[pl.ds(s * MROWS + q * (MROWS // 4),
                                        MROWS // 4)])
        plsc.subcore_barrier()

        pltpu.sync_copy(vA.at[g, pl.ds(s * CHUNKS, CHUNKS)], vidx_v)
        pltpu.sync_copy(eA.at[g, pl.ds(s * CHUNKS, CHUNKS)], eidx_v)

        def _off_row(j, _):
            for k in range(8):
                sl = pl.ds(k * 16, 16)
                vidx_v[j, sl] = vidx_v[j, sl] + cN
            return 0
        lax.fori_loop(0, CHUNKS, _off_row, 0)

        _gs_pipeline(h2, vidx_v, eidx_v, acc_sh, rows_a, rows_b, sem_a, sem_b)
        plsc.subcore_barrier()

        pltpu.sync_copy(acc_sh.at[pl.ds(s * MROWS, MROWS)],
                        acc_out.at[c, g, pl.ds(s * MROWS, MROWS)])
        return 0

    lax.fori_loop(0, 3, _group, 0)


def _make_sc_v2e():
    mesh = plsc.VectorSubcoreMesh(core_axis_name="c", subcore_axis_name="s")
    return pl.kernel(
        _sc_v2e_body,
        mesh=mesh,
        compiler_params=pltpu.CompilerParams(use_tc_tiling_on_sc=False),
        out_type=jax.ShapeDtypeStruct((2, 3, M_PAD, 128), jnp.float32),
        scratch_types=[
            pltpu.VMEM((CHUNKS, 128), jnp.int32),
            pltpu.VMEM((CHUNKS, 128), jnp.int32),
            pltpu.VMEM((128, 128), jnp.float32),
            pltpu.VMEM((128, 128), jnp.float32),
            pltpu.VMEM((MROWS // 4, 128), jnp.float32),
            pltpu.VMEM_SHARED((M_PAD, 128), jnp.float32),
            pltpu.SemaphoreType.DMA,
            pltpu.SemaphoreType.DMA,
        ],
    )


HROWS = M_PAD * 16 // 128


def _sc_cnt_body(eA, cnt_out, eidx_v, hist_v, sem):
    c = lax.axis_index("c")
    s = lax.axis_index("s")
    del sem
    lanes = lax.iota(jnp.int32, 16)
    ones16 = jnp.ones((16,), jnp.float32)

    for g in range(3):
        def _zero(j, _):
            for k in range(8):
                hist_v[j, k * 16:(k + 1) * 16] = jnp.zeros((16,), jnp.float32)
            return 0
        lax.fori_loop(0, HROWS, _zero, 0)

        pltpu.sync_copy(
            eA.at[g, pl.ds((c * NTILES + s) * CCHUNKS, CCHUNKS)], eidx_v)

        def _chunk(j, _):
            for k in range(8):
                ev = eidx_v[j, k * 16:(k + 1) * 16]
                flat = ev * 16 + lanes
                plsc.addupdate_scatter(
                    hist_v, [lax.shift_right_logical(flat, 7), flat & 127],
                    ones16)
            return 0
        lax.fori_loop(0, CCHUNKS, _chunk, 0)

        pltpu.sync_copy(hist_v, cnt_out.at[c, s, g])


def _make_sc_cnt():
    mesh = plsc.VectorSubcoreMesh(core_axis_name="c", subcore_axis_name="s")
    return pl.kernel(
        _sc_cnt_body,
        mesh=mesh,
        compiler_params=pltpu.CompilerParams(needs_layout_passes=False),
        out_type=jax.ShapeDtypeStruct((2, NTILES, 3, HROWS, 128), jnp.float32),
        scratch_types=[
            pltpu.VMEM((CCHUNKS, 128), jnp.int32),
            pltpu.VMEM((HROWS, 128), jnp.float32),
            pltpu.SemaphoreType.DMA,
        ],
    )


def _kb_body(acc_ref, cnt_ref, w_ref, a_ref, z_ref):
    ch = jnp.sum(cnt_ref[...], axis=(0, 1, 2))
    ri = lax.broadcasted_iota(jnp.int32, (MBLK, MBLK // 8), 0)
    ci = lax.broadcasted_iota(jnp.int32, (MBLK, MBLK // 8), 1)
    U = (ri // 8 == ci).astype(jnp.float32)
    chr_ = jnp.dot(U, ch, preferred_element_type=jnp.float32)
    li = lax.broadcasted_iota(jnp.int32, (MBLK, 128), 1)
    ii = lax.broadcasted_iota(jnp.int32, (MBLK, 128), 0)
    sel = (li // 16) == (ii % 8)
    cnt = jnp.sum(jnp.where(sel, chr_, 0.0), axis=1, keepdims=True)
    cnt = jnp.maximum(cnt, 1.0)
    y0 = acc_ref[0, 0] / cnt
    y1 = acc_ref[1, 0] / cnt
    alpha = (jnp.dot(y0, w_ref[0], preferred_element_type=jnp.float32)
             + jnp.dot(y1, w_ref[1], preferred_element_type=jnp.float32))
    t = jnp.where(alpha >= 0, alpha, 0.2 * alpha)
    t = jnp.clip(t, 0.0, 5.0) * a_ref[pl.program_id(0), 0]
    z0 = y0 * t
    z1 = y1 * t
    z_ref[0, 0, 0] = z0[:, :64]
    z_ref[0, 1, 0] = z0[:, 64:]
    z_ref[1, 0, 0] = z1[:, :64]
    z_ref[1, 1, 0] = z1[:, 64:]


def _stage3(Acc, Cnt, w_e, avec):
    mb = M_PAD // MBLK
    return pl.pallas_call(
        _kb_body,
        grid=(3, mb),
        in_specs=[
            pl.BlockSpec((2, 1, MBLK, 128), lambda g, m: (0, g, m, 0)),
            pl.BlockSpec((2, NTILES, 1, MBLK * 16 // 128, 128),
                         lambda g, m: (0, 0, g, m, 0)),
            pl.BlockSpec((2, 128, 1), lambda g, m: (0, 0, 0)),
            pl.BlockSpec(memory_space=pltpu.SMEM),
        ],
        out_specs=pl.BlockSpec((2, 2, 1, MBLK, 64), lambda g, m: (0, 0, g, m, 0)),
        out_shape=jax.ShapeDtypeStruct((2, 2, 3, M_PAD, 64), jnp.float32),
    )(Acc, Cnt, w_e.reshape(2, 128, 1), avec)


def _sc_e2v_body(zflat, vC, eC, x_out, vidx_v, eidx_v, rows_a, rows_b,
                 rows_c, rows_d, zbuf_v, xacc_sh, sem_a, sem_b, sem_c, sem_d):
    c = lax.axis_index("c")
    s = lax.axis_index("s")
    XROWS = N_PAD // NTILES

    def _zero_row(j, _):
        for k in range(4):
            zbuf_v[j, k * 16:(k + 1) * 16] = jnp.zeros((16,), jnp.float32)
        return 0
    lax.fori_loop(0, XROWS // 4, _zero_row, 0)

    for p in range(2):
        for q in range(4):
            pltpu.sync_copy(
                zbuf_v, xacc_sh.at[pl.ds(s * XROWS + q * (XROWS // 4),
                                         XROWS // 4)])
        plsc.subcore_barrier()

        for g in range(3):
            plane = ((c * 2 + p) * 3 + g) * M_PAD
            pltpu.sync_copy(vC.at[g, pl.ds(s * CHUNKS, CHUNKS)], vidx_v)
            pltpu.sync_copy(eC.at[g, pl.ds(s * CHUNKS, CHUNKS)], eidx_v)

            def _off_row(j, _):
                for k in range(8):
                    sl = pl.ds(k * 16, 16)
                    eidx_v[j, sl] = eidx_v[j, sl] + plane
                return 0
            lax.fori_loop(0, CHUNKS, _off_row, 0)

            _gs_pipeline4(zflat, eidx_v, vidx_v, xacc_sh,
                          [rows_a, rows_b, rows_c, rows_d],
                          [sem_a, sem_b, sem_c, sem_d])
        plsc.subcore_barrier()

        pltpu.sync_copy(xacc_sh.at[pl.ds(s * XROWS, XROWS)],
                        x_out.at[c, p, pl.ds(s * XROWS, XROWS)])
        plsc.subcore_barrier()


def _make_sc_e2v():
    mesh = plsc.VectorSubcoreMesh(core_axis_name="c", subcore_axis_name="s")
    return pl.kernel(
        _sc_e2v_body,
        mesh=mesh,
        compiler_params=pltpu.CompilerParams(use_tc_tiling_on_sc=False),
        out_type=jax.ShapeDtypeStruct((2, 2, N_PAD, 64), jnp.float32),
        scratch_types=[
            pltpu.VMEM((CHUNKS, 128), jnp.int32),
            pltpu.VMEM((CHUNKS, 128), jnp.int32),
            pltpu.VMEM((128, 64), jnp.float32),
            pltpu.VMEM((128, 64), jnp.float32),
            pltpu.VMEM((128, 64), jnp.float32),
            pltpu.VMEM((128, 64), jnp.float32),
            pltpu.VMEM((N_PAD // NTILES // 4, 64), jnp.float32),
            pltpu.VMEM_SHARED((N_PAD, 64), jnp.float32),
            pltpu.SemaphoreType.DMA,
            pltpu.SemaphoreType.DMA,
            pltpu.SemaphoreType.DMA,
            pltpu.SemaphoreType.DMA,
        ],
    )


def _kd_body(x_ref, o_ref):
    for c in range(2):
        for p in range(2):
            xq = x_ref[c, p]
            lo = (c * 2 + p) * 64
            o_ref[:, lo:lo + 64] = jnp.where(xq >= 0, xq, 0.01 * xq)


def _stage5(Xout):
    nb = N // RBLK
    return pl.pallas_call(
        _kd_body,
        grid=(nb,),
        in_specs=[pl.BlockSpec((2, 2, RBLK, 64), lambda i: (0, 0, i, 0))],
        out_specs=pl.BlockSpec((RBLK, D), lambda i: (i, 0)),
        out_shape=jax.ShapeDtypeStruct((N, D), jnp.float32),
    )(Xout)


def _pad_idx(v, e):
    ar = jnp.arange(E_PAD - E, dtype=jnp.int32)
    vA = jnp.concatenate([v, ar % 64])
    eA = jnp.concatenate([e, M + ar % (M_PAD - M)])
    vC = jnp.concatenate([v, N + ar % (N_PAD - N)])
    return vA, eA, vC


def kernel(X, v_hier, e_hier, v_cooc, e_cooc, v_cite, e_cite,
           W, b, gamma, beta, w_e, a1, a2, a3):
    groups = [(v_hier, e_hier), (v_cooc, e_cooc), (v_cite, e_cite)]
    vAs, eAs, vCs = [], [], []
    for v, e in groups:
        vA, eA, vC = _pad_idx(v, e)
        vAs.append(vA)
        eAs.append(eA)
        vCs.append(vC)
    vA3 = jnp.stack(vAs).reshape(3, E_PAD // 128, 128)
    eA3 = jnp.stack(eAs).reshape(3, E_PAD // 128, 128)
    vC3 = jnp.stack(vCs).reshape(3, E_PAD // 128, 128)

    Cnt = _make_sc_cnt()(eA3)
    H2 = _stage1(X, W, b, gamma, beta).reshape(2 * N, 128)
    Acc = _make_sc_v2e()(H2, vA3, eA3)
    avec = jnp.concatenate([a1.ravel(), a2.ravel(), a3.ravel()]).reshape(3, 1)
    Z = _stage3(Acc, Cnt, w_e, avec)
    Xout = _make_sc_e2v()(Z.reshape(12 * M_PAD, 64), vC3, eA3)
    Xo = _stage5(Xout)
    a = jnp.concatenate([a1.ravel(), a2.ravel(), a3.ravel()])
    return (Xo, a)
